# Initial kernel scaffold; baseline (speedup 1.0000x reference)
import jax
import jax.numpy as jnp
from jax.experimental import pallas as pl


def kernel(x, edge_index, batch, W_enc, b_enc, eps, W1, b1, g1, be1, W2, b2, g2, be2, W_cls, b_cls):
    raise NotImplementedError("write your pallas kernel here")


# speedup vs baseline: 6.1721x; 6.1721x over previous
import functools

import jax
import jax.numpy as jnp
from jax import lax
from jax.experimental import pallas as pl
from jax.experimental.pallas import tpu as pltpu
from jax.experimental.pallas import tpu_sc as plsc

N = 50000
E = 1600000
H = 64
HH = 32
L = 3
G = 128
C = 2

RB = 2000
NB = N // RB

NPAD = 51200
TRASH = N
TILE_ROWS = NPAD // 16
OCH = 49
IR = 16
EPAD = OCH * IR * 128 * 16
ER = EPAD // 128
PTR = OCH * IR


def _sc_segment_sum(h_lo, h_hi, src2, dst2, zblk):
    mesh = plsc.VectorSubcoreMesh(core_axis_name="c", subcore_axis_name="s",
                                  num_cores=2, num_subcores=16)

    @functools.partial(
        pl.kernel,
        out_type=jax.ShapeDtypeStruct((2, NPAD, HH), jnp.float32),
        mesh=mesh,
        compiler_params=pltpu.CompilerParams(use_tc_tiling_on_sc=False),
        scratch_types=[
            pltpu.VMEM((IR, 128), jnp.int32),
            pltpu.VMEM((IR, 128), jnp.int32),
            pltpu.VMEM((128, HH), jnp.float32),
            pltpu.VMEM((128, HH), jnp.float32),
            pltpu.VMEM_SHARED((NPAD, HH), jnp.float32),
            pltpu.SemaphoreType.DMA,
        ],
    )
    def k(hlo_hbm, hhi_hbm, src_hbm, dst_hbm, z_hbm, out_hbm,
          src_v, dst_v, rows_v, buf_v, acc, sem):
        c = lax.axis_index("c")
        s = lax.axis_index("s")

        pltpu.sync_copy(z_hbm, buf_v)

        def zloop(i, carry):
            pltpu.sync_copy(buf_v, acc.at[pl.ds(s * TILE_ROWS + i * 128, 128)])
            return carry
        lax.fori_loop(0, TILE_ROWS // 128, zloop, 0)
        plsc.subcore_barrier()

        def outer(i, carry):
            r0 = s * PTR + i * IR
            pltpu.sync_copy(src_hbm.at[pl.ds(r0, IR)], src_v)
            pltpu.sync_copy(dst_hbm.at[pl.ds(r0, IR)], dst_v)

            def inner(j, icarry):
                @pl.when(c == 0)
                def _():
                    pltpu.async_copy(hlo_hbm.at[src_v.at[j]], rows_v, sem).wait()

                @pl.when(c == 1)
                def _():
                    pltpu.async_copy(hhi_hbm.at[src_v.at[j]], rows_v, sem).wait()

                pltpu.sync_copy(rows_v, acc.at[dst_v.at[j]], add=True)
                return icarry
            lax.fori_loop(0, IR, inner, 0)
            return carry
        lax.fori_loop(0, OCH, outer, 0)
        plsc.subcore_barrier()

        def wloop(i, carry):
            b0 = s * TILE_ROWS + i * 128
            pltpu.sync_copy(acc.at[pl.ds(b0, 128)], buf_v)
            pltpu.sync_copy(buf_v, out_hbm.at[c, pl.ds(b0, 128)])
            return carry
        lax.fori_loop(0, TILE_ROWS // 128, wloop, 0)

    return k(h_lo, h_hi, src2, dst2, zblk)


def _encoder(x, W_enc, b_enc):
    def body(x_ref, w_ref, b_ref, lo_ref, hi_ref):
        hblk = x_ref[...] * w_ref[...] + b_ref[...]
        lo_ref[...] = hblk[:, :HH]
        hi_ref[...] = hblk[:, HH:]

    return pl.pallas_call(
        body,
        grid=(NB,),
        in_specs=[
            pl.BlockSpec((RB, 1), lambda i: (i, 0)),
            pl.BlockSpec((1, H), lambda i: (0, 0)),
            pl.BlockSpec((1, H), lambda i: (0, 0)),
        ],
        out_specs=[pl.BlockSpec((RB, HH), lambda i: (i, 0))] * 2,
        out_shape=[jax.ShapeDtypeStruct((N, HH), jnp.float32)] * 2,
    )(x, W_enc, b_enc.reshape(1, H))


def _pass_a(h_lo, h_hi, agg, eps_l, W1_l, b1_l):
    def body(hlo_ref, hhi_ref, agg_ref, eps_ref, w_ref, b_ref, u_ref, st_ref, acc):
        i = pl.program_id(0)
        h = jnp.concatenate([hlo_ref[...], hhi_ref[...]], axis=1)
        a = jnp.concatenate([agg_ref[0], agg_ref[1]], axis=1)
        t = eps_ref[0, 0] * h + a
        u = jnp.dot(t, w_ref[...], preferred_element_type=jnp.float32) + b_ref[...]
        u_ref[...] = u

        @pl.when(i == 0)
        def _():
            acc[...] = jnp.zeros_like(acc)
        acc[0:1, :] += jnp.sum(u, axis=0, keepdims=True)
        acc[1:2, :] += jnp.sum(u * u, axis=0, keepdims=True)

        @pl.when(i == NB - 1)
        def _():
            st_ref[...] = acc[...]

    return pl.pallas_call(
        body,
        grid=(NB,),
        in_specs=[
            pl.BlockSpec((RB, HH), lambda i: (i, 0)),
            pl.BlockSpec((RB, HH), lambda i: (i, 0)),
            pl.BlockSpec((2, RB, HH), lambda i: (0, i, 0)),
            pl.BlockSpec((1, 1), lambda i: (0, 0)),
            pl.BlockSpec((H, H), lambda i: (0, 0)),
            pl.BlockSpec((1, H), lambda i: (0, 0)),
        ],
        out_specs=[
            pl.BlockSpec((RB, H), lambda i: (i, 0)),
            pl.BlockSpec((2, H), lambda i: (0, 0)),
        ],
        out_shape=[
            jax.ShapeDtypeStruct((N, H), jnp.float32),
            jax.ShapeDtypeStruct((2, H), jnp.float32),
        ],
        scratch_shapes=[pltpu.VMEM((2, H), jnp.float32)],
    )(h_lo, h_hi, agg, (1.0 + eps_l).reshape(1, 1), W1_l, b1_l.reshape(1, H))


def _pass_b(u, su, g_l, be_l, W2_l, b2_l):
    def body(u_ref, st_ref, g_ref, be_ref, w_ref, b_ref, v_ref, sv_ref, acc):
        i = pl.program_id(0)
        mean = st_ref[0:1, :] * (1.0 / N)
        var = st_ref[1:2, :] * (1.0 / N) - mean * mean
        un = (u_ref[...] - mean) * (lax.rsqrt(var + 1e-5) * g_ref[...]) + be_ref[...]
        un = jnp.maximum(un, 0.0)
        v = jnp.dot(un, w_ref[...], preferred_element_type=jnp.float32) + b_ref[...]
        v_ref[...] = v

        @pl.when(i == 0)
        def _():
            acc[...] = jnp.zeros_like(acc)
        acc[0:1, :] += jnp.sum(v, axis=0, keepdims=True)
        acc[1:2, :] += jnp.sum(v * v, axis=0, keepdims=True)

        @pl.when(i == NB - 1)
        def _():
            sv_ref[...] = acc[...]

    return pl.pallas_call(
        body,
        grid=(NB,),
        in_specs=[
            pl.BlockSpec((RB, H), lambda i: (i, 0)),
            pl.BlockSpec((2, H), lambda i: (0, 0)),
            pl.BlockSpec((1, H), lambda i: (0, 0)),
            pl.BlockSpec((1, H), lambda i: (0, 0)),
            pl.BlockSpec((H, H), lambda i: (0, 0)),
            pl.BlockSpec((1, H), lambda i: (0, 0)),
        ],
        out_specs=[
            pl.BlockSpec((RB, H), lambda i: (i, 0)),
            pl.BlockSpec((2, H), lambda i: (0, 0)),
        ],
        out_shape=[
            jax.ShapeDtypeStruct((N, H), jnp.float32),
            jax.ShapeDtypeStruct((2, H), jnp.float32),
        ],
        scratch_shapes=[pltpu.VMEM((2, H), jnp.float32)],
    )(u, su, g_l.reshape(1, H), be_l.reshape(1, H), W2_l, b2_l.reshape(1, H))


def _pass_c(v, sv, g_l, be_l):
    def body(v_ref, st_ref, g_ref, be_ref, lo_ref, hi_ref):
        mean = st_ref[0:1, :] * (1.0 / N)
        var = st_ref[1:2, :] * (1.0 / N) - mean * mean
        hn = (v_ref[...] - mean) * (lax.rsqrt(var + 1e-5) * g_ref[...]) + be_ref[...]
        hn = jnp.maximum(hn, 0.0)
        lo_ref[...] = hn[:, :HH]
        hi_ref[...] = hn[:, HH:]

    return pl.pallas_call(
        body,
        grid=(NB,),
        in_specs=[
            pl.BlockSpec((RB, H), lambda i: (i, 0)),
            pl.BlockSpec((2, H), lambda i: (0, 0)),
            pl.BlockSpec((1, H), lambda i: (0, 0)),
            pl.BlockSpec((1, H), lambda i: (0, 0)),
        ],
        out_specs=[pl.BlockSpec((RB, HH), lambda i: (i, 0))] * 2,
        out_shape=[jax.ShapeDtypeStruct((N, HH), jnp.float32)] * 2,
    )(v, sv, g_l.reshape(1, H), be_l.reshape(1, H))


def _pool_cls(h_lo, h_hi, batch3, W_cls, b_cls):
    def body(lo_ref, hi_ref, b_ref, w_ref, bc_ref, o_ref, acc):
        i = pl.program_id(0)

        @pl.when(i == 0)
        def _():
            acc[...] = jnp.zeros_like(acc)

        h = jnp.concatenate(
            [lo_ref[...], hi_ref[...], jnp.ones((RB, 1), jnp.float32)], axis=1)
        bv = b_ref[0, 0, :]
        onehot = (bv[:, None] ==
                  lax.broadcasted_iota(jnp.int32, (RB, G), 1)).astype(jnp.float32)
        acc[...] += lax.dot_general(
            onehot, h, (((0,), (0,)), ((), ())),
            preferred_element_type=jnp.float32)

        @pl.when(i == NB - 1)
        def _():
            cnt = jnp.maximum(acc[:, H:H + 1], 1.0)
            pooled = acc[:, :H] / cnt
            o_ref[...] = jnp.dot(pooled, w_ref[...],
                                 preferred_element_type=jnp.float32) + bc_ref[...]

    return pl.pallas_call(
        body,
        grid=(NB,),
        in_specs=[
            pl.BlockSpec((RB, HH), lambda i: (i, 0)),
            pl.BlockSpec((RB, HH), lambda i: (i, 0)),
            pl.BlockSpec((1, 1, RB), lambda i: (i, 0, 0)),
            pl.BlockSpec((H, C), lambda i: (0, 0)),
            pl.BlockSpec((1, C), lambda i: (0, 0)),
        ],
        out_specs=pl.BlockSpec((G, C), lambda i: (0, 0)),
        out_shape=jax.ShapeDtypeStruct((G, C), jnp.float32),
        scratch_shapes=[pltpu.VMEM((G, H + 1), jnp.float32)],
    )(h_lo, h_hi, batch3, W_cls, b_cls.reshape(1, C))


def kernel(x, edge_index, batch, W_enc, b_enc, eps, W1, b1, g1, be1,
           W2, b2, g2, be2, W_cls, b_cls):
    src = edge_index[0].astype(jnp.int32)
    dst = edge_index[1].astype(jnp.int32)
    padn = EPAD - E
    src2 = jnp.concatenate([src, jnp.zeros((padn,), jnp.int32)]).reshape(ER, 128)
    dst2 = jnp.concatenate([dst, jnp.full((padn,), TRASH, jnp.int32)]).reshape(ER, 128)
    zblk = jnp.zeros((128, HH), jnp.float32)
    batch3 = batch.astype(jnp.int32).reshape(NB, 1, RB)

    h_lo, h_hi = _encoder(x, W_enc, b_enc)
    for l in range(L):
        agg = _sc_segment_sum(h_lo, h_hi, src2, dst2, zblk)
        u, su = _pass_a(h_lo, h_hi, agg, eps[l], W1[l], b1[l])
        v, sv = _pass_b(u, su, g1[l], be1[l], W2[l], b2[l])
        h_lo, h_hi = _pass_c(v, sv, g2[l], be2[l])
    return _pool_cls(h_lo, h_hi, batch3, W_cls, b_cls)


# speedup vs baseline: 7.8713x; 1.2753x over previous
import functools

import jax
import jax.numpy as jnp
from jax import lax
from jax.experimental import pallas as pl
from jax.experimental.pallas import tpu as pltpu
from jax.experimental.pallas import tpu_sc as plsc

N = 50000
E = 1600000
H = 64
HH = 32
L = 3
G = 128
C = 2

RB = 2000
NB = N // RB

NPAD = 51200
TRASH = N
TILE_ROWS = NPAD // 16
PTR = 784
EPAD = PTR * 128 * 16
ER = EPAD // 128
CH = 2
NCH = PTR // CH


def _sc_segment_sum(h2, src3, dst2, zblk):
    mesh = plsc.VectorSubcoreMesh(core_axis_name="c", subcore_axis_name="s",
                                  num_cores=2, num_subcores=16)

    @functools.partial(
        pl.kernel,
        out_type=jax.ShapeDtypeStruct((2, NPAD, HH), jnp.float32),
        mesh=mesh,
        compiler_params=pltpu.CompilerParams(use_tc_tiling_on_sc=False),
        scratch_types=[
            pltpu.VMEM((2, CH, 128), jnp.int32),
            pltpu.VMEM((2, CH, 128), jnp.int32),
            pltpu.VMEM((2, CH, 128, HH), jnp.float32),
            pltpu.VMEM((128, HH), jnp.float32),
            pltpu.VMEM_SHARED((NPAD, HH), jnp.float32),
            pltpu.SemaphoreType.DMA((2,)),
        ],
    )
    def k(h2_hbm, src_hbm, dst_hbm, z_hbm, out_hbm,
          idx_s, idx_d, rows, buf_v, acc, sems):
        c = lax.axis_index("c")
        s = lax.axis_index("s")

        pltpu.sync_copy(z_hbm, buf_v)

        def zloop(i, carry):
            pltpu.sync_copy(buf_v, acc.at[pl.ds(s * TILE_ROWS + i * 128, 128)])
            return carry
        lax.fori_loop(0, TILE_ROWS // 128, zloop, 0)
        plsc.subcore_barrier()

        def load_fire(kc):
            p = lax.rem(kc, 2)
            r0 = s * PTR + kc * CH
            pltpu.sync_copy(src_hbm.at[c, pl.ds(r0, CH)], idx_s.at[p])
            pltpu.sync_copy(dst_hbm.at[pl.ds(r0, CH)], idx_d.at[p])

            def fj(j, carry):
                pltpu.async_copy(
                    h2_hbm.at[idx_s.at[p, j]], rows.at[p, j], sems.at[p])
                return carry
            lax.fori_loop(0, CH, fj, 0)

        def drain_scatter(kc):
            p = lax.rem(kc, 2)

            def wj(j, carry):
                pltpu.make_async_copy(
                    h2_hbm.at[idx_s.at[p, j]], rows.at[p, j],
                    sems.at[p]).wait()
                return carry
            lax.fori_loop(0, CH, wj, 0)

            def sj(j, carry):
                pltpu.sync_copy(rows.at[p, j], acc.at[idx_d.at[p, j]],
                                add=True)
                return carry
            lax.fori_loop(0, CH, sj, 0)

        load_fire(0)

        def body(kc, carry):
            @pl.when(kc < NCH - 1)
            def _():
                load_fire(kc + 1)
            drain_scatter(kc)
            return carry
        lax.fori_loop(0, NCH, body, 0)
        plsc.subcore_barrier()

        def wloop(i, carry):
            b0 = s * TILE_ROWS + i * 128
            pltpu.sync_copy(acc.at[pl.ds(b0, 128)], buf_v)
            pltpu.sync_copy(buf_v, out_hbm.at[c, pl.ds(b0, 128)])
            return carry
        lax.fori_loop(0, TILE_ROWS // 128, wloop, 0)

    return k(h2, src3, dst2, zblk)


def _encoder(x, W_enc, b_enc):
    def body(x_ref, w_ref, b_ref, h2_ref):
        i = pl.program_id(0)
        hblk = x_ref[...] * w_ref[...] + b_ref[...]
        h2_ref[...] = jnp.where(i < NB, hblk[:, :HH], hblk[:, HH:])

    return pl.pallas_call(
        body,
        grid=(2 * NB,),
        in_specs=[
            pl.BlockSpec((RB, 1), lambda i: (i % NB, 0)),
            pl.BlockSpec((1, H), lambda i: (0, 0)),
            pl.BlockSpec((1, H), lambda i: (0, 0)),
        ],
        out_specs=pl.BlockSpec((RB, HH), lambda i: (i, 0)),
        out_shape=jax.ShapeDtypeStruct((2 * N, HH), jnp.float32),
    )(x, W_enc, b_enc.reshape(1, H))


def _pass_a(h2, agg, eps_l, W1_l, b1_l):
    def body(hlo_ref, hhi_ref, agg_ref, eps_ref, w_ref, b_ref, u_ref, st_ref, acc):
        i = pl.program_id(0)
        h = jnp.concatenate([hlo_ref[...], hhi_ref[...]], axis=1)
        a = jnp.concatenate([agg_ref[0], agg_ref[1]], axis=1)
        t = eps_ref[0, 0] * h + a
        u = jnp.dot(t, w_ref[...], preferred_element_type=jnp.float32) + b_ref[...]
        u_ref[...] = u

        @pl.when(i == 0)
        def _():
            acc[...] = jnp.zeros_like(acc)
        acc[0:1, :] += jnp.sum(u, axis=0, keepdims=True)
        acc[1:2, :] += jnp.sum(u * u, axis=0, keepdims=True)

        @pl.when(i == NB - 1)
        def _():
            st_ref[...] = acc[...]

    return pl.pallas_call(
        body,
        grid=(NB,),
        in_specs=[
            pl.BlockSpec((RB, HH), lambda i: (i, 0)),
            pl.BlockSpec((RB, HH), lambda i: (NB + i, 0)),
            pl.BlockSpec((2, RB, HH), lambda i: (0, i, 0)),
            pl.BlockSpec((1, 1), lambda i: (0, 0)),
            pl.BlockSpec((H, H), lambda i: (0, 0)),
            pl.BlockSpec((1, H), lambda i: (0, 0)),
        ],
        out_specs=[
            pl.BlockSpec((RB, H), lambda i: (i, 0)),
            pl.BlockSpec((2, H), lambda i: (0, 0)),
        ],
        out_shape=[
            jax.ShapeDtypeStruct((N, H), jnp.float32),
            jax.ShapeDtypeStruct((2, H), jnp.float32),
        ],
        scratch_shapes=[pltpu.VMEM((2, H), jnp.float32)],
    )(h2, h2, agg, (1.0 + eps_l).reshape(1, 1), W1_l, b1_l.reshape(1, H))


def _pass_b(u, su, g_l, be_l, W2_l, b2_l):
    def body(u_ref, st_ref, g_ref, be_ref, w_ref, b_ref, v_ref, sv_ref, acc):
        i = pl.program_id(0)
        mean = st_ref[0:1, :] * (1.0 / N)
        var = st_ref[1:2, :] * (1.0 / N) - mean * mean
        un = (u_ref[...] - mean) * (lax.rsqrt(var + 1e-5) * g_ref[...]) + be_ref[...]
        un = jnp.maximum(un, 0.0)
        v = jnp.dot(un, w_ref[...], preferred_element_type=jnp.float32) + b_ref[...]
        v_ref[...] = v

        @pl.when(i == 0)
        def _():
            acc[...] = jnp.zeros_like(acc)
        acc[0:1, :] += jnp.sum(v, axis=0, keepdims=True)
        acc[1:2, :] += jnp.sum(v * v, axis=0, keepdims=True)

        @pl.when(i == NB - 1)
        def _():
            sv_ref[...] = acc[...]

    return pl.pallas_call(
        body,
        grid=(NB,),
        in_specs=[
            pl.BlockSpec((RB, H), lambda i: (i, 0)),
            pl.BlockSpec((2, H), lambda i: (0, 0)),
            pl.BlockSpec((1, H), lambda i: (0, 0)),
            pl.BlockSpec((1, H), lambda i: (0, 0)),
            pl.BlockSpec((H, H), lambda i: (0, 0)),
            pl.BlockSpec((1, H), lambda i: (0, 0)),
        ],
        out_specs=[
            pl.BlockSpec((RB, H), lambda i: (i, 0)),
            pl.BlockSpec((2, H), lambda i: (0, 0)),
        ],
        out_shape=[
            jax.ShapeDtypeStruct((N, H), jnp.float32),
            jax.ShapeDtypeStruct((2, H), jnp.float32),
        ],
        scratch_shapes=[pltpu.VMEM((2, H), jnp.float32)],
    )(u, su, g_l.reshape(1, H), be_l.reshape(1, H), W2_l, b2_l.reshape(1, H))


def _pass_c(v, sv, g_l, be_l):
    def body(v_ref, st_ref, g_ref, be_ref, h2_ref):
        i = pl.program_id(0)
        mean = st_ref[0:1, :] * (1.0 / N)
        var = st_ref[1:2, :] * (1.0 / N) - mean * mean
        hn = (v_ref[...] - mean) * (lax.rsqrt(var + 1e-5) * g_ref[...]) + be_ref[...]
        hn = jnp.maximum(hn, 0.0)
        h2_ref[...] = jnp.where(i < NB, hn[:, :HH], hn[:, HH:])

    return pl.pallas_call(
        body,
        grid=(2 * NB,),
        in_specs=[
            pl.BlockSpec((RB, H), lambda i: (i % NB, 0)),
            pl.BlockSpec((2, H), lambda i: (0, 0)),
            pl.BlockSpec((1, H), lambda i: (0, 0)),
            pl.BlockSpec((1, H), lambda i: (0, 0)),
        ],
        out_specs=pl.BlockSpec((RB, HH), lambda i: (i, 0)),
        out_shape=jax.ShapeDtypeStruct((2 * N, HH), jnp.float32),
    )(v, sv, g_l.reshape(1, H), be_l.reshape(1, H))


def _pool_cls(h2, batch3, W_cls, b_cls):
    def body(lo_ref, hi_ref, b_ref, w_ref, bc_ref, o_ref, acc):
        i = pl.program_id(0)

        @pl.when(i == 0)
        def _():
            acc[...] = jnp.zeros_like(acc)

        h = jnp.concatenate(
            [lo_ref[...], hi_ref[...], jnp.ones((RB, 1), jnp.float32)], axis=1)
        bv = b_ref[0, 0, :]
        onehot = (bv[:, None] ==
                  lax.broadcasted_iota(jnp.int32, (RB, G), 1)).astype(jnp.float32)
        acc[...] += lax.dot_general(
            onehot, h, (((0,), (0,)), ((), ())),
            preferred_element_type=jnp.float32)

        @pl.when(i == NB - 1)
        def _():
            cnt = jnp.maximum(acc[:, H:H + 1], 1.0)
            pooled = acc[:, :H] / cnt
            o_ref[...] = jnp.dot(pooled, w_ref[...],
                                 preferred_element_type=jnp.float32) + bc_ref[...]

    return pl.pallas_call(
        body,
        grid=(NB,),
        in_specs=[
            pl.BlockSpec((RB, HH), lambda i: (i, 0)),
            pl.BlockSpec((RB, HH), lambda i: (NB + i, 0)),
            pl.BlockSpec((1, 1, RB), lambda i: (i, 0, 0)),
            pl.BlockSpec((H, C), lambda i: (0, 0)),
            pl.BlockSpec((1, C), lambda i: (0, 0)),
        ],
        out_specs=pl.BlockSpec((G, C), lambda i: (0, 0)),
        out_shape=jax.ShapeDtypeStruct((G, C), jnp.float32),
        scratch_shapes=[pltpu.VMEM((G, H + 1), jnp.float32)],
    )(h2, h2, batch3, W_cls, b_cls.reshape(1, C))


def kernel(x, edge_index, batch, W_enc, b_enc, eps, W1, b1, g1, be1,
           W2, b2, g2, be2, W_cls, b_cls):
    src = edge_index[0].astype(jnp.int32)
    dst = edge_index[1].astype(jnp.int32)
    padn = EPAD - E
    src2 = jnp.concatenate([src, jnp.zeros((padn,), jnp.int32)]).reshape(ER, 128)
    src3 = jnp.stack([src2, src2 + N])
    dst2 = jnp.concatenate([dst, jnp.full((padn,), TRASH, jnp.int32)]).reshape(ER, 128)
    zblk = jnp.zeros((128, HH), jnp.float32)
    batch3 = batch.astype(jnp.int32).reshape(NB, 1, RB)

    h2 = _encoder(x, W_enc, b_enc)
    for l in range(L):
        agg = _sc_segment_sum(h2, src3, dst2, zblk)
        u, su = _pass_a(h2, agg, eps[l], W1[l], b1[l])
        v, sv = _pass_b(u, su, g1[l], be1[l], W2[l], b2[l])
        h2 = _pass_c(v, sv, g2[l], be2[l])
    return _pool_cls(h2, batch3, W_cls, b_cls)


# speedup vs baseline: 9.2200x; 1.1713x over previous
import functools

import jax
import jax.numpy as jnp
from jax import lax
from jax.experimental import pallas as pl
from jax.experimental.pallas import tpu as pltpu
from jax.experimental.pallas import tpu_sc as plsc

N = 50000
E = 1600000
H = 64
HH = 32
L = 3
G = 128
C = 2

RB = 2000
NB = N // RB

NPAD = 51200
TRASH = N
TILE_ROWS = NPAD // 16
PTR = 784
EPAD = PTR * 128 * 16
ER = EPAD // 128
CH = 2
NCH = PTR // CH


def _sc_segment_sum(h2, src3, dst2, zblk):
    mesh = plsc.VectorSubcoreMesh(core_axis_name="c", subcore_axis_name="s",
                                  num_cores=2, num_subcores=16)

    @functools.partial(
        pl.kernel,
        out_type=jax.ShapeDtypeStruct((2, NPAD, HH), jnp.float32),
        mesh=mesh,
        compiler_params=pltpu.CompilerParams(use_tc_tiling_on_sc=False),
        scratch_types=[
            pltpu.VMEM((3, CH, 128), jnp.int32),
            pltpu.VMEM((3, CH, 128), jnp.int32),
            pltpu.VMEM((3, CH, 128, HH), jnp.float32),
            pltpu.VMEM_SHARED((NPAD, HH), jnp.float32),
            pltpu.SemaphoreType.DMA((3,)),
            pltpu.SemaphoreType.DMA((3,)),
        ],
    )
    def k(h2_hbm, src_hbm, dst_hbm, z_hbm, out_hbm,
          idx_s, idx_d, rows, acc, gsem, ssem):
        c = lax.axis_index("c")
        s = lax.axis_index("s")

        pltpu.sync_copy(z_hbm, rows.at[0, 0])

        def zloop(i, carry):
            pltpu.sync_copy(rows.at[0, 0],
                            acc.at[pl.ds(s * TILE_ROWS + i * 128, 128)])
            return carry
        lax.fori_loop(0, TILE_ROWS // 128, zloop, 0)
        plsc.subcore_barrier()

        def load_fire(kc):
            p = lax.rem(kc, 3)
            r0 = s * PTR + kc * CH
            pltpu.sync_copy(src_hbm.at[c, pl.ds(r0, CH)], idx_s.at[p])
            pltpu.sync_copy(dst_hbm.at[pl.ds(r0, CH)], idx_d.at[p])

            def fj(j, carry):
                pltpu.async_copy(
                    h2_hbm.at[idx_s.at[p, j]], rows.at[p, j], gsem.at[p])
                return carry
            lax.fori_loop(0, CH, fj, 0)

        def wait_gathers(kc):
            p = lax.rem(kc, 3)

            def wj(j, carry):
                pltpu.make_async_copy(
                    h2_hbm.at[idx_s.at[p, j]], rows.at[p, j],
                    gsem.at[p]).wait()
                return carry
            lax.fori_loop(0, CH, wj, 0)

        def fire_scatters(kc):
            p = lax.rem(kc, 3)

            def sj(j, carry):
                pltpu.async_copy(rows.at[p, j], acc.at[idx_d.at[p, j]],
                                 ssem.at[p], add=True)
                return carry
            lax.fori_loop(0, CH, sj, 0)

        def wait_scatters(kc):
            p = lax.rem(kc, 3)

            def wj(j, carry):
                pltpu.make_async_copy(rows.at[p, j], acc.at[idx_d.at[p, j]],
                                      ssem.at[p]).wait()
                return carry
            lax.fori_loop(0, CH, wj, 0)

        lax.fori_loop(0, 2, lambda i, cr: (load_fire(i), cr)[1], 0)

        def body(kc, carry):
            wait_gathers(kc)
            fire_scatters(kc)

            @pl.when(kc >= 1)
            def _():
                wait_scatters(kc - 1)

            @pl.when(kc + 2 < NCH)
            def _():
                load_fire(kc + 2)
            return carry
        lax.fori_loop(0, NCH, body, 0)
        wait_scatters(NCH - 1)
        plsc.subcore_barrier()

        def wloop(i, carry):
            b0 = s * TILE_ROWS + i * 128
            pltpu.sync_copy(acc.at[pl.ds(b0, 128)], rows.at[0, 0])
            pltpu.sync_copy(rows.at[0, 0], out_hbm.at[c, pl.ds(b0, 128)])
            return carry
        lax.fori_loop(0, TILE_ROWS // 128, wloop, 0)

    return k(h2, src3, dst2, zblk)


def _encoder(x, W_enc, b_enc):
    def body(x_ref, w_ref, b_ref, h2_ref):
        i = pl.program_id(0)
        hblk = x_ref[...] * w_ref[...] + b_ref[...]
        h2_ref[...] = jnp.where(i < NB, hblk[:, :HH], hblk[:, HH:])

    return pl.pallas_call(
        body,
        grid=(2 * NB,),
        in_specs=[
            pl.BlockSpec((RB, 1), lambda i: (i % NB, 0)),
            pl.BlockSpec((1, H), lambda i: (0, 0)),
            pl.BlockSpec((1, H), lambda i: (0, 0)),
        ],
        out_specs=pl.BlockSpec((RB, HH), lambda i: (i, 0)),
        out_shape=jax.ShapeDtypeStruct((2 * N, HH), jnp.float32),
    )(x, W_enc, b_enc.reshape(1, H))


def _pass_a(h2, agg, eps_l, W1_l, b1_l):
    def body(hlo_ref, hhi_ref, agg_ref, eps_ref, w_ref, b_ref, u_ref, st_ref, acc):
        i = pl.program_id(0)
        h = jnp.concatenate([hlo_ref[...], hhi_ref[...]], axis=1)
        a = jnp.concatenate([agg_ref[0], agg_ref[1]], axis=1)
        t = eps_ref[0, 0] * h + a
        u = jnp.dot(t, w_ref[...], preferred_element_type=jnp.float32) + b_ref[...]
        u_ref[...] = u

        @pl.when(i == 0)
        def _():
            acc[...] = jnp.zeros_like(acc)
        acc[0:1, :] += jnp.sum(u, axis=0, keepdims=True)
        acc[1:2, :] += jnp.sum(u * u, axis=0, keepdims=True)

        @pl.when(i == NB - 1)
        def _():
            st_ref[...] = acc[...]

    return pl.pallas_call(
        body,
        grid=(NB,),
        in_specs=[
            pl.BlockSpec((RB, HH), lambda i: (i, 0)),
            pl.BlockSpec((RB, HH), lambda i: (NB + i, 0)),
            pl.BlockSpec((2, RB, HH), lambda i: (0, i, 0)),
            pl.BlockSpec((1, 1), lambda i: (0, 0)),
            pl.BlockSpec((H, H), lambda i: (0, 0)),
            pl.BlockSpec((1, H), lambda i: (0, 0)),
        ],
        out_specs=[
            pl.BlockSpec((RB, H), lambda i: (i, 0)),
            pl.BlockSpec((2, H), lambda i: (0, 0)),
        ],
        out_shape=[
            jax.ShapeDtypeStruct((N, H), jnp.float32),
            jax.ShapeDtypeStruct((2, H), jnp.float32),
        ],
        scratch_shapes=[pltpu.VMEM((2, H), jnp.float32)],
    )(h2, h2, agg, (1.0 + eps_l).reshape(1, 1), W1_l, b1_l.reshape(1, H))


def _pass_b(u, su, g_l, be_l, W2_l, b2_l):
    def body(u_ref, st_ref, g_ref, be_ref, w_ref, b_ref, v_ref, sv_ref, acc):
        i = pl.program_id(0)
        mean = st_ref[0:1, :] * (1.0 / N)
        var = st_ref[1:2, :] * (1.0 / N) - mean * mean
        un = (u_ref[...] - mean) * (lax.rsqrt(var + 1e-5) * g_ref[...]) + be_ref[...]
        un = jnp.maximum(un, 0.0)
        v = jnp.dot(un, w_ref[...], preferred_element_type=jnp.float32) + b_ref[...]
        v_ref[...] = v

        @pl.when(i == 0)
        def _():
            acc[...] = jnp.zeros_like(acc)
        acc[0:1, :] += jnp.sum(v, axis=0, keepdims=True)
        acc[1:2, :] += jnp.sum(v * v, axis=0, keepdims=True)

        @pl.when(i == NB - 1)
        def _():
            sv_ref[...] = acc[...]

    return pl.pallas_call(
        body,
        grid=(NB,),
        in_specs=[
            pl.BlockSpec((RB, H), lambda i: (i, 0)),
            pl.BlockSpec((2, H), lambda i: (0, 0)),
            pl.BlockSpec((1, H), lambda i: (0, 0)),
            pl.BlockSpec((1, H), lambda i: (0, 0)),
            pl.BlockSpec((H, H), lambda i: (0, 0)),
            pl.BlockSpec((1, H), lambda i: (0, 0)),
        ],
        out_specs=[
            pl.BlockSpec((RB, H), lambda i: (i, 0)),
            pl.BlockSpec((2, H), lambda i: (0, 0)),
        ],
        out_shape=[
            jax.ShapeDtypeStruct((N, H), jnp.float32),
            jax.ShapeDtypeStruct((2, H), jnp.float32),
        ],
        scratch_shapes=[pltpu.VMEM((2, H), jnp.float32)],
    )(u, su, g_l.reshape(1, H), be_l.reshape(1, H), W2_l, b2_l.reshape(1, H))


def _pass_c(v, sv, g_l, be_l):
    def body(v_ref, st_ref, g_ref, be_ref, h2_ref):
        i = pl.program_id(0)
        mean = st_ref[0:1, :] * (1.0 / N)
        var = st_ref[1:2, :] * (1.0 / N) - mean * mean
        hn = (v_ref[...] - mean) * (lax.rsqrt(var + 1e-5) * g_ref[...]) + be_ref[...]
        hn = jnp.maximum(hn, 0.0)
        h2_ref[...] = jnp.where(i < NB, hn[:, :HH], hn[:, HH:])

    return pl.pallas_call(
        body,
        grid=(2 * NB,),
        in_specs=[
            pl.BlockSpec((RB, H), lambda i: (i % NB, 0)),
            pl.BlockSpec((2, H), lambda i: (0, 0)),
            pl.BlockSpec((1, H), lambda i: (0, 0)),
            pl.BlockSpec((1, H), lambda i: (0, 0)),
        ],
        out_specs=pl.BlockSpec((RB, HH), lambda i: (i, 0)),
        out_shape=jax.ShapeDtypeStruct((2 * N, HH), jnp.float32),
    )(v, sv, g_l.reshape(1, H), be_l.reshape(1, H))


def _pool_cls(h2, batch3, W_cls, b_cls):
    def body(lo_ref, hi_ref, b_ref, w_ref, bc_ref, o_ref, acc):
        i = pl.program_id(0)

        @pl.when(i == 0)
        def _():
            acc[...] = jnp.zeros_like(acc)

        h = jnp.concatenate(
            [lo_ref[...], hi_ref[...], jnp.ones((RB, 1), jnp.float32)], axis=1)
        bv = b_ref[0, 0, :]
        onehot = (bv[:, None] ==
                  lax.broadcasted_iota(jnp.int32, (RB, G), 1)).astype(jnp.float32)
        acc[...] += lax.dot_general(
            onehot, h, (((0,), (0,)), ((), ())),
            preferred_element_type=jnp.float32)

        @pl.when(i == NB - 1)
        def _():
            cnt = jnp.maximum(acc[:, H:H + 1], 1.0)
            pooled = acc[:, :H] / cnt
            o_ref[...] = jnp.dot(pooled, w_ref[...],
                                 preferred_element_type=jnp.float32) + bc_ref[...]

    return pl.pallas_call(
        body,
        grid=(NB,),
        in_specs=[
            pl.BlockSpec((RB, HH), lambda i: (i, 0)),
            pl.BlockSpec((RB, HH), lambda i: (NB + i, 0)),
            pl.BlockSpec((1, 1, RB), lambda i: (i, 0, 0)),
            pl.BlockSpec((H, C), lambda i: (0, 0)),
            pl.BlockSpec((1, C), lambda i: (0, 0)),
        ],
        out_specs=pl.BlockSpec((G, C), lambda i: (0, 0)),
        out_shape=jax.ShapeDtypeStruct((G, C), jnp.float32),
        scratch_shapes=[pltpu.VMEM((G, H + 1), jnp.float32)],
    )(h2, h2, batch3, W_cls, b_cls.reshape(1, C))


def kernel(x, edge_index, batch, W_enc, b_enc, eps, W1, b1, g1, be1,
           W2, b2, g2, be2, W_cls, b_cls):
    src = edge_index[0].astype(jnp.int32)
    dst = edge_index[1].astype(jnp.int32)
    padn = EPAD - E
    src2 = jnp.concatenate([src, jnp.zeros((padn,), jnp.int32)]).reshape(ER, 128)
    src3 = jnp.stack([src2, src2 + N])
    dst2 = jnp.concatenate([dst, jnp.full((padn,), TRASH, jnp.int32)]).reshape(ER, 128)
    zblk = jnp.zeros((128, HH), jnp.float32)
    batch3 = batch.astype(jnp.int32).reshape(NB, 1, RB)

    h2 = _encoder(x, W_enc, b_enc)
    for l in range(L):
        agg = _sc_segment_sum(h2, src3, dst2, zblk)
        u, su = _pass_a(h2, agg, eps[l], W1[l], b1[l])
        v, sv = _pass_b(u, su, g1[l], be1[l], W2[l], b2[l])
        h2 = _pass_c(v, sv, g2[l], be2[l])
    return _pool_cls(h2, batch3, W_cls, b_cls)


# speedup vs baseline: 12.0104x; 1.3026x over previous
import functools

import jax
import jax.numpy as jnp
from jax import lax
from jax.experimental import pallas as pl
from jax.experimental.pallas import tpu as pltpu
from jax.experimental.pallas import tpu_sc as plsc

N = 50000
E = 1600000
H = 64
HH = 32
L = 3
G = 128
C = 2

RB = 2000
NB = N // RB

NPAD = 51200
TRASH = N
TILE_ROWS = NPAD // 16
PTR = 784
EPAD = PTR * 128 * 16
ER = EPAD // 128
CH = 2
NCH = PTR // CH


def _sc_segment_sum(h2, idxc, zblk):
    mesh = plsc.VectorSubcoreMesh(core_axis_name="c", subcore_axis_name="s",
                                  num_cores=2, num_subcores=16)

    @functools.partial(
        pl.kernel,
        out_type=jax.ShapeDtypeStruct((2, NPAD, HH), jnp.float32),
        mesh=mesh,
        compiler_params=pltpu.CompilerParams(use_tc_tiling_on_sc=False),
        scratch_types=[
            pltpu.VMEM((6, CH, 2, 128), jnp.int32),
            pltpu.VMEM((3, CH, 128, HH), jnp.float32),
            pltpu.VMEM_SHARED((NPAD, HH), jnp.float32),
            pltpu.SemaphoreType.DMA((6,)),
            pltpu.SemaphoreType.DMA((3,)),
            pltpu.SemaphoreType.DMA((3,)),
        ],
    )
    def k(h2_hbm, idx_hbm, z_hbm, out_hbm,
          idx, rows, acc, isem, gsem, ssem):
        c = lax.axis_index("c")
        s = lax.axis_index("s")

        pltpu.sync_copy(z_hbm, rows.at[0, 0])

        def zloop(i, carry):
            pltpu.sync_copy(rows.at[0, 0],
                            acc.at[pl.ds(s * TILE_ROWS + i * 128, 128)])
            return carry
        lax.fori_loop(0, TILE_ROWS // 128, zloop, 0)
        plsc.subcore_barrier()

        def fire_idx(kc):
            q = lax.rem(kc, 6)
            r0 = s * PTR + kc * CH
            pltpu.async_copy(idx_hbm.at[c, pl.ds(r0, CH)], idx.at[q],
                             isem.at[q])

        def wait_idx(kc):
            q = lax.rem(kc, 6)
            r0 = s * PTR + kc * CH
            pltpu.make_async_copy(idx_hbm.at[c, pl.ds(r0, CH)], idx.at[q],
                                  isem.at[q]).wait()

        def fire_gathers(kc):
            p = lax.rem(kc, 3)
            q = lax.rem(kc, 6)

            def fj(j, carry):
                pltpu.async_copy(
                    h2_hbm.at[idx.at[q, j, 0]], rows.at[p, j], gsem.at[p])
                return carry
            lax.fori_loop(0, CH, fj, 0)

        def wait_gathers(kc):
            p = lax.rem(kc, 3)
            q = lax.rem(kc, 6)

            def wj(j, carry):
                pltpu.make_async_copy(
                    h2_hbm.at[idx.at[q, j, 0]], rows.at[p, j],
                    gsem.at[p]).wait()
                return carry
            lax.fori_loop(0, CH, wj, 0)

        def fire_scatters(kc):
            p = lax.rem(kc, 3)
            q = lax.rem(kc, 6)

            def sj(j, carry):
                pltpu.async_copy(rows.at[p, j], acc.at[idx.at[q, j, 1]],
                                 ssem.at[p], add=True)
                return carry
            lax.fori_loop(0, CH, sj, 0)

        def wait_scatters(kc):
            p = lax.rem(kc, 3)
            q = lax.rem(kc, 6)

            def wj(j, carry):
                pltpu.make_async_copy(rows.at[p, j], acc.at[idx.at[q, j, 1]],
                                      ssem.at[p]).wait()
                return carry
            lax.fori_loop(0, CH, wj, 0)

        lax.fori_loop(0, 4, lambda i, cr: (fire_idx(i), cr)[1], 0)

        def prol(i, cr):
            wait_idx(i)
            fire_gathers(i)
            return cr
        lax.fori_loop(0, 2, prol, 0)

        def body(kc, carry):
            @pl.when(kc >= 1)
            def _():
                wait_scatters(kc - 1)

            @pl.when(kc + 2 < NCH)
            def _():
                wait_idx(kc + 2)
                fire_gathers(kc + 2)

            wait_gathers(kc)
            fire_scatters(kc)

            @pl.when(kc + 4 < NCH)
            def _():
                fire_idx(kc + 4)
            return carry
        lax.fori_loop(0, NCH, body, 0)
        wait_scatters(NCH - 1)
        plsc.subcore_barrier()

        def wloop(i, carry):
            b0 = s * TILE_ROWS + i * 128
            pltpu.sync_copy(acc.at[pl.ds(b0, 128)], rows.at[0, 0])
            pltpu.sync_copy(rows.at[0, 0], out_hbm.at[c, pl.ds(b0, 128)])
            return carry
        lax.fori_loop(0, TILE_ROWS // 128, wloop, 0)

    return k(h2, idxc, zblk)


def _encoder(x, W_enc, b_enc):
    def body(x_ref, w_ref, b_ref, h2_ref):
        i = pl.program_id(0)
        hblk = x_ref[...] * w_ref[...] + b_ref[...]
        h2_ref[...] = jnp.where(i < NB, hblk[:, :HH], hblk[:, HH:])

    return pl.pallas_call(
        body,
        grid=(2 * NB,),
        in_specs=[
            pl.BlockSpec((RB, 1), lambda i: (i % NB, 0)),
            pl.BlockSpec((1, H), lambda i: (0, 0)),
            pl.BlockSpec((1, H), lambda i: (0, 0)),
        ],
        out_specs=pl.BlockSpec((RB, HH), lambda i: (i, 0)),
        out_shape=jax.ShapeDtypeStruct((2 * N, HH), jnp.float32),
    )(x, W_enc, b_enc.reshape(1, H))


def _pass_a(h2, agg, eps_l, W1_l, b1_l):
    def body(hlo_ref, hhi_ref, agg_ref, eps_ref, w_ref, b_ref, u_ref, st_ref, acc):
        i = pl.program_id(0)
        h = jnp.concatenate([hlo_ref[...], hhi_ref[...]], axis=1)
        a = jnp.concatenate([agg_ref[0], agg_ref[1]], axis=1)
        t = eps_ref[0, 0] * h + a
        u = jnp.dot(t, w_ref[...], preferred_element_type=jnp.float32) + b_ref[...]
        u_ref[...] = u

        @pl.when(i == 0)
        def _():
            acc[...] = jnp.zeros_like(acc)
        acc[0:1, :] += jnp.sum(u, axis=0, keepdims=True)
        acc[1:2, :] += jnp.sum(u * u, axis=0, keepdims=True)

        @pl.when(i == NB - 1)
        def _():
            st_ref[...] = acc[...]

    return pl.pallas_call(
        body,
        grid=(NB,),
        in_specs=[
            pl.BlockSpec((RB, HH), lambda i: (i, 0)),
            pl.BlockSpec((RB, HH), lambda i: (NB + i, 0)),
            pl.BlockSpec((2, RB, HH), lambda i: (0, i, 0)),
            pl.BlockSpec((1, 1), lambda i: (0, 0)),
            pl.BlockSpec((H, H), lambda i: (0, 0)),
            pl.BlockSpec((1, H), lambda i: (0, 0)),
        ],
        out_specs=[
            pl.BlockSpec((RB, H), lambda i: (i, 0)),
            pl.BlockSpec((2, H), lambda i: (0, 0)),
        ],
        out_shape=[
            jax.ShapeDtypeStruct((N, H), jnp.float32),
            jax.ShapeDtypeStruct((2, H), jnp.float32),
        ],
        scratch_shapes=[pltpu.VMEM((2, H), jnp.float32)],
    )(h2, h2, agg, (1.0 + eps_l).reshape(1, 1), W1_l, b1_l.reshape(1, H))


def _pass_b(u, su, g_l, be_l, W2_l, b2_l):
    def body(u_ref, st_ref, g_ref, be_ref, w_ref, b_ref, v_ref, sv_ref, acc):
        i = pl.program_id(0)
        mean = st_ref[0:1, :] * (1.0 / N)
        var = st_ref[1:2, :] * (1.0 / N) - mean * mean
        un = (u_ref[...] - mean) * (lax.rsqrt(var + 1e-5) * g_ref[...]) + be_ref[...]
        un = jnp.maximum(un, 0.0)
        v = jnp.dot(un, w_ref[...], preferred_element_type=jnp.float32) + b_ref[...]
        v_ref[...] = v

        @pl.when(i == 0)
        def _():
            acc[...] = jnp.zeros_like(acc)
        acc[0:1, :] += jnp.sum(v, axis=0, keepdims=True)
        acc[1:2, :] += jnp.sum(v * v, axis=0, keepdims=True)

        @pl.when(i == NB - 1)
        def _():
            sv_ref[...] = acc[...]

    return pl.pallas_call(
        body,
        grid=(NB,),
        in_specs=[
            pl.BlockSpec((RB, H), lambda i: (i, 0)),
            pl.BlockSpec((2, H), lambda i: (0, 0)),
            pl.BlockSpec((1, H), lambda i: (0, 0)),
            pl.BlockSpec((1, H), lambda i: (0, 0)),
            pl.BlockSpec((H, H), lambda i: (0, 0)),
            pl.BlockSpec((1, H), lambda i: (0, 0)),
        ],
        out_specs=[
            pl.BlockSpec((RB, H), lambda i: (i, 0)),
            pl.BlockSpec((2, H), lambda i: (0, 0)),
        ],
        out_shape=[
            jax.ShapeDtypeStruct((N, H), jnp.float32),
            jax.ShapeDtypeStruct((2, H), jnp.float32),
        ],
        scratch_shapes=[pltpu.VMEM((2, H), jnp.float32)],
    )(u, su, g_l.reshape(1, H), be_l.reshape(1, H), W2_l, b2_l.reshape(1, H))


def _pass_c(v, sv, g_l, be_l):
    def body(v_ref, st_ref, g_ref, be_ref, h2_ref):
        i = pl.program_id(0)
        mean = st_ref[0:1, :] * (1.0 / N)
        var = st_ref[1:2, :] * (1.0 / N) - mean * mean
        hn = (v_ref[...] - mean) * (lax.rsqrt(var + 1e-5) * g_ref[...]) + be_ref[...]
        hn = jnp.maximum(hn, 0.0)
        h2_ref[...] = jnp.where(i < NB, hn[:, :HH], hn[:, HH:])

    return pl.pallas_call(
        body,
        grid=(2 * NB,),
        in_specs=[
            pl.BlockSpec((RB, H), lambda i: (i % NB, 0)),
            pl.BlockSpec((2, H), lambda i: (0, 0)),
            pl.BlockSpec((1, H), lambda i: (0, 0)),
            pl.BlockSpec((1, H), lambda i: (0, 0)),
        ],
        out_specs=pl.BlockSpec((RB, HH), lambda i: (i, 0)),
        out_shape=jax.ShapeDtypeStruct((2 * N, HH), jnp.float32),
    )(v, sv, g_l.reshape(1, H), be_l.reshape(1, H))


def _pool_cls(h2, batch3, W_cls, b_cls):
    def body(lo_ref, hi_ref, b_ref, w_ref, bc_ref, o_ref, acc):
        i = pl.program_id(0)

        @pl.when(i == 0)
        def _():
            acc[...] = jnp.zeros_like(acc)

        h = jnp.concatenate(
            [lo_ref[...], hi_ref[...], jnp.ones((RB, 1), jnp.float32)], axis=1)
        bv = b_ref[0, 0, :]
        onehot = (bv[:, None] ==
                  lax.broadcasted_iota(jnp.int32, (RB, G), 1)).astype(jnp.float32)
        acc[...] += lax.dot_general(
            onehot, h, (((0,), (0,)), ((), ())),
            preferred_element_type=jnp.float32)

        @pl.when(i == NB - 1)
        def _():
            cnt = jnp.maximum(acc[:, H:H + 1], 1.0)
            pooled = acc[:, :H] / cnt
            o_ref[...] = jnp.dot(pooled, w_ref[...],
                                 preferred_element_type=jnp.float32) + bc_ref[...]

    return pl.pallas_call(
        body,
        grid=(NB,),
        in_specs=[
            pl.BlockSpec((RB, HH), lambda i: (i, 0)),
            pl.BlockSpec((RB, HH), lambda i: (NB + i, 0)),
            pl.BlockSpec((1, 1, RB), lambda i: (i, 0, 0)),
            pl.BlockSpec((H, C), lambda i: (0, 0)),
            pl.BlockSpec((1, C), lambda i: (0, 0)),
        ],
        out_specs=pl.BlockSpec((G, C), lambda i: (0, 0)),
        out_shape=jax.ShapeDtypeStruct((G, C), jnp.float32),
        scratch_shapes=[pltpu.VMEM((G, H + 1), jnp.float32)],
    )(h2, h2, batch3, W_cls, b_cls.reshape(1, C))


def kernel(x, edge_index, batch, W_enc, b_enc, eps, W1, b1, g1, be1,
           W2, b2, g2, be2, W_cls, b_cls):
    src = edge_index[0].astype(jnp.int32)
    dst = edge_index[1].astype(jnp.int32)
    padn = EPAD - E
    src2 = jnp.concatenate([src, jnp.zeros((padn,), jnp.int32)]).reshape(ER, 128)
    dst2 = jnp.concatenate([dst, jnp.full((padn,), TRASH, jnp.int32)]).reshape(ER, 128)
    idxc = jnp.stack([jnp.stack([src2, dst2], axis=1),
                      jnp.stack([src2 + N, dst2], axis=1)])
    zblk = jnp.zeros((128, HH), jnp.float32)
    batch3 = batch.astype(jnp.int32).reshape(NB, 1, RB)

    h2 = _encoder(x, W_enc, b_enc)
    for l in range(L):
        agg = _sc_segment_sum(h2, idxc, zblk)
        u, su = _pass_a(h2, agg, eps[l], W1[l], b1[l])
        v, sv = _pass_b(u, su, g1[l], be1[l], W2[l], b2[l])
        h2 = _pass_c(v, sv, g2[l], be2[l])
    return _pool_cls(h2, batch3, W_cls, b_cls)


# speedup vs baseline: 12.0286x; 1.0015x over previous
import functools

import jax
import jax.numpy as jnp
from jax import lax
from jax.experimental import pallas as pl
from jax.experimental.pallas import tpu as pltpu
from jax.experimental.pallas import tpu_sc as plsc

N = 50000
E = 1600000
H = 64
HH = 32
L = 3
G = 128
C = 2

RB = 2000
NB = N // RB

NPAD = 51200
TRASH = N
TILE_ROWS = NPAD // 16
PTR = 784
EPAD = PTR * 128 * 16
ER = EPAD // 128
CH = 2
NCH = PTR // CH


def _sc_segment_sum(h2, idxc, zblk):
    mesh = plsc.VectorSubcoreMesh(core_axis_name="c", subcore_axis_name="s",
                                  num_cores=2, num_subcores=16)

    @functools.partial(
        pl.kernel,
        out_type=jax.ShapeDtypeStruct((2, NPAD, HH), jnp.float32),
        mesh=mesh,
        compiler_params=pltpu.CompilerParams(use_tc_tiling_on_sc=False),
        scratch_types=[
            pltpu.VMEM((6, 2 * CH, 128), jnp.int32),
            pltpu.VMEM((3, CH, 128, HH), jnp.float32),
            pltpu.VMEM_SHARED((NPAD, HH), jnp.float32),
            pltpu.SemaphoreType.DMA((6,)),
            pltpu.SemaphoreType.DMA((3,)),
            pltpu.SemaphoreType.DMA((3,)),
        ],
    )
    def k(h2_hbm, idx_hbm, z_hbm, out_hbm,
          idx, rows, acc, isem, gsem, ssem):
        c = lax.axis_index("c")
        s = lax.axis_index("s")

        pltpu.sync_copy(z_hbm, rows.at[0, 0])

        def zloop(i, carry):
            pltpu.sync_copy(rows.at[0, 0],
                            acc.at[pl.ds(s * TILE_ROWS + i * 128, 128)])
            return carry
        lax.fori_loop(0, TILE_ROWS // 128, zloop, 0)
        plsc.subcore_barrier()

        def fire_idx(kc):
            q = lax.rem(kc, 6)
            r0 = 2 * (s * PTR + kc * CH)
            pltpu.async_copy(idx_hbm.at[c, pl.ds(r0, 2 * CH)], idx.at[q],
                             isem.at[q])

        def wait_idx(kc):
            q = lax.rem(kc, 6)
            r0 = 2 * (s * PTR + kc * CH)
            pltpu.make_async_copy(idx_hbm.at[c, pl.ds(r0, 2 * CH)], idx.at[q],
                                  isem.at[q]).wait()

        def fire_gathers(kc):
            p = lax.rem(kc, 3)
            q = lax.rem(kc, 6)

            def fj(j, carry):
                pltpu.async_copy(
                    h2_hbm.at[idx.at[q, 2 * j]], rows.at[p, j], gsem.at[p])
                return carry
            lax.fori_loop(0, CH, fj, 0)

        def wait_gathers(kc):
            p = lax.rem(kc, 3)
            q = lax.rem(kc, 6)

            def wj(j, carry):
                pltpu.make_async_copy(
                    h2_hbm.at[idx.at[q, 2 * j]], rows.at[p, j],
                    gsem.at[p]).wait()
                return carry
            lax.fori_loop(0, CH, wj, 0)

        def fire_scatters(kc):
            p = lax.rem(kc, 3)
            q = lax.rem(kc, 6)

            def sj(j, carry):
                pltpu.async_copy(rows.at[p, j], acc.at[idx.at[q, 2 * j + 1]],
                                 ssem.at[p], add=True)
                return carry
            lax.fori_loop(0, CH, sj, 0)

        def wait_scatters(kc):
            p = lax.rem(kc, 3)
            q = lax.rem(kc, 6)

            def wj(j, carry):
                pltpu.make_async_copy(rows.at[p, j], acc.at[idx.at[q, 2 * j + 1]],
                                      ssem.at[p]).wait()
                return carry
            lax.fori_loop(0, CH, wj, 0)

        lax.fori_loop(0, 4, lambda i, cr: (fire_idx(i), cr)[1], 0)

        def prol(i, cr):
            wait_idx(i)
            fire_gathers(i)
            return cr
        lax.fori_loop(0, 2, prol, 0)

        def body(kc, carry):
            @pl.when(kc >= 1)
            def _():
                wait_scatters(kc - 1)

            @pl.when(kc + 2 < NCH)
            def _():
                wait_idx(kc + 2)
                fire_gathers(kc + 2)

            wait_gathers(kc)
            fire_scatters(kc)

            @pl.when(kc + 4 < NCH)
            def _():
                fire_idx(kc + 4)
            return carry
        lax.fori_loop(0, NCH, body, 0)
        wait_scatters(NCH - 1)
        plsc.subcore_barrier()

        def wloop(i, carry):
            b0 = s * TILE_ROWS + i * 128
            pltpu.sync_copy(acc.at[pl.ds(b0, 128)], rows.at[0, 0])
            pltpu.sync_copy(rows.at[0, 0], out_hbm.at[c, pl.ds(b0, 128)])
            return carry
        lax.fori_loop(0, TILE_ROWS // 128, wloop, 0)

    return k(h2, idxc, zblk)


def _encoder(x, W_enc, b_enc):
    def body(x_ref, w_ref, b_ref, h2_ref):
        i = pl.program_id(0)
        hblk = x_ref[...] * w_ref[...] + b_ref[...]
        h2_ref[...] = jnp.where(i < NB, hblk[:, :HH], hblk[:, HH:])

    return pl.pallas_call(
        body,
        grid=(2 * NB,),
        in_specs=[
            pl.BlockSpec((RB, 1), lambda i: (i % NB, 0)),
            pl.BlockSpec((1, H), lambda i: (0, 0)),
            pl.BlockSpec((1, H), lambda i: (0, 0)),
        ],
        out_specs=pl.BlockSpec((RB, HH), lambda i: (i, 0)),
        out_shape=jax.ShapeDtypeStruct((2 * N, HH), jnp.float32),
    )(x, W_enc, b_enc.reshape(1, H))


def _pass_a(h2, agg, eps_l, W1_l, b1_l):
    def body(hlo_ref, hhi_ref, agg_ref, eps_ref, w_ref, b_ref, u_ref, st_ref, acc):
        i = pl.program_id(0)
        h = jnp.concatenate([hlo_ref[...], hhi_ref[...]], axis=1)
        a = jnp.concatenate([agg_ref[0], agg_ref[1]], axis=1)
        t = eps_ref[0, 0] * h + a
        u = jnp.dot(t, w_ref[...], preferred_element_type=jnp.float32) + b_ref[...]
        u_ref[...] = u

        @pl.when(i == 0)
        def _():
            acc[...] = jnp.zeros_like(acc)
        acc[0:1, :] += jnp.sum(u, axis=0, keepdims=True)
        acc[1:2, :] += jnp.sum(u * u, axis=0, keepdims=True)

        @pl.when(i == NB - 1)
        def _():
            st_ref[...] = acc[...]

    return pl.pallas_call(
        body,
        grid=(NB,),
        in_specs=[
            pl.BlockSpec((RB, HH), lambda i: (i, 0)),
            pl.BlockSpec((RB, HH), lambda i: (NB + i, 0)),
            pl.BlockSpec((2, RB, HH), lambda i: (0, i, 0)),
            pl.BlockSpec((1, 1), lambda i: (0, 0)),
            pl.BlockSpec((H, H), lambda i: (0, 0)),
            pl.BlockSpec((1, H), lambda i: (0, 0)),
        ],
        out_specs=[
            pl.BlockSpec((RB, H), lambda i: (i, 0)),
            pl.BlockSpec((2, H), lambda i: (0, 0)),
        ],
        out_shape=[
            jax.ShapeDtypeStruct((N, H), jnp.float32),
            jax.ShapeDtypeStruct((2, H), jnp.float32),
        ],
        scratch_shapes=[pltpu.VMEM((2, H), jnp.float32)],
    )(h2, h2, agg, (1.0 + eps_l).reshape(1, 1), W1_l, b1_l.reshape(1, H))


def _pass_b(u, su, g_l, be_l, W2_l, b2_l):
    def body(u_ref, st_ref, g_ref, be_ref, w_ref, b_ref, v_ref, sv_ref, acc):
        i = pl.program_id(0)
        mean = st_ref[0:1, :] * (1.0 / N)
        var = st_ref[1:2, :] * (1.0 / N) - mean * mean
        un = (u_ref[...] - mean) * (lax.rsqrt(var + 1e-5) * g_ref[...]) + be_ref[...]
        un = jnp.maximum(un, 0.0)
        v = jnp.dot(un, w_ref[...], preferred_element_type=jnp.float32) + b_ref[...]
        v_ref[...] = v

        @pl.when(i == 0)
        def _():
            acc[...] = jnp.zeros_like(acc)
        acc[0:1, :] += jnp.sum(v, axis=0, keepdims=True)
        acc[1:2, :] += jnp.sum(v * v, axis=0, keepdims=True)

        @pl.when(i == NB - 1)
        def _():
            sv_ref[...] = acc[...]

    return pl.pallas_call(
        body,
        grid=(NB,),
        in_specs=[
            pl.BlockSpec((RB, H), lambda i: (i, 0)),
            pl.BlockSpec((2, H), lambda i: (0, 0)),
            pl.BlockSpec((1, H), lambda i: (0, 0)),
            pl.BlockSpec((1, H), lambda i: (0, 0)),
            pl.BlockSpec((H, H), lambda i: (0, 0)),
            pl.BlockSpec((1, H), lambda i: (0, 0)),
        ],
        out_specs=[
            pl.BlockSpec((RB, H), lambda i: (i, 0)),
            pl.BlockSpec((2, H), lambda i: (0, 0)),
        ],
        out_shape=[
            jax.ShapeDtypeStruct((N, H), jnp.float32),
            jax.ShapeDtypeStruct((2, H), jnp.float32),
        ],
        scratch_shapes=[pltpu.VMEM((2, H), jnp.float32)],
    )(u, su, g_l.reshape(1, H), be_l.reshape(1, H), W2_l, b2_l.reshape(1, H))


def _pass_c(v, sv, g_l, be_l):
    def body(v_ref, st_ref, g_ref, be_ref, h2_ref):
        i = pl.program_id(0)
        mean = st_ref[0:1, :] * (1.0 / N)
        var = st_ref[1:2, :] * (1.0 / N) - mean * mean
        hn = (v_ref[...] - mean) * (lax.rsqrt(var + 1e-5) * g_ref[...]) + be_ref[...]
        hn = jnp.maximum(hn, 0.0)
        h2_ref[...] = jnp.where(i < NB, hn[:, :HH], hn[:, HH:])

    return pl.pallas_call(
        body,
        grid=(2 * NB,),
        in_specs=[
            pl.BlockSpec((RB, H), lambda i: (i % NB, 0)),
            pl.BlockSpec((2, H), lambda i: (0, 0)),
            pl.BlockSpec((1, H), lambda i: (0, 0)),
            pl.BlockSpec((1, H), lambda i: (0, 0)),
        ],
        out_specs=pl.BlockSpec((RB, HH), lambda i: (i, 0)),
        out_shape=jax.ShapeDtypeStruct((2 * N, HH), jnp.float32),
    )(v, sv, g_l.reshape(1, H), be_l.reshape(1, H))


def _pool_cls(h2, batch3, W_cls, b_cls):
    def body(lo_ref, hi_ref, b_ref, w_ref, bc_ref, o_ref, acc):
        i = pl.program_id(0)

        @pl.when(i == 0)
        def _():
            acc[...] = jnp.zeros_like(acc)

        h = jnp.concatenate(
            [lo_ref[...], hi_ref[...], jnp.ones((RB, 1), jnp.float32)], axis=1)
        bv = b_ref[0, 0, :]
        onehot = (bv[:, None] ==
                  lax.broadcasted_iota(jnp.int32, (RB, G), 1)).astype(jnp.float32)
        acc[...] += lax.dot_general(
            onehot, h, (((0,), (0,)), ((), ())),
            preferred_element_type=jnp.float32)

        @pl.when(i == NB - 1)
        def _():
            cnt = jnp.maximum(acc[:, H:H + 1], 1.0)
            pooled = acc[:, :H] / cnt
            o_ref[...] = jnp.dot(pooled, w_ref[...],
                                 preferred_element_type=jnp.float32) + bc_ref[...]

    return pl.pallas_call(
        body,
        grid=(NB,),
        in_specs=[
            pl.BlockSpec((RB, HH), lambda i: (i, 0)),
            pl.BlockSpec((RB, HH), lambda i: (NB + i, 0)),
            pl.BlockSpec((1, 1, RB), lambda i: (i, 0, 0)),
            pl.BlockSpec((H, C), lambda i: (0, 0)),
            pl.BlockSpec((1, C), lambda i: (0, 0)),
        ],
        out_specs=pl.BlockSpec((G, C), lambda i: (0, 0)),
        out_shape=jax.ShapeDtypeStruct((G, C), jnp.float32),
        scratch_shapes=[pltpu.VMEM((G, H + 1), jnp.float32)],
    )(h2, h2, batch3, W_cls, b_cls.reshape(1, C))


def kernel(x, edge_index, batch, W_enc, b_enc, eps, W1, b1, g1, be1,
           W2, b2, g2, be2, W_cls, b_cls):
    src = edge_index[0].astype(jnp.int32)
    dst = edge_index[1].astype(jnp.int32)
    padn = EPAD - E
    src2 = jnp.concatenate([src, jnp.zeros((padn,), jnp.int32)]).reshape(ER, 128)
    dst2 = jnp.concatenate([dst, jnp.full((padn,), TRASH, jnp.int32)]).reshape(ER, 128)
    idxc = jnp.stack(
        [jnp.stack([src2, dst2], axis=1).reshape(2 * ER, 128),
         jnp.stack([src2 + N, dst2], axis=1).reshape(2 * ER, 128)])
    zblk = jnp.zeros((128, HH), jnp.float32)
    batch3 = batch.astype(jnp.int32).reshape(NB, 1, RB)

    h2 = _encoder(x, W_enc, b_enc)
    for l in range(L):
        agg = _sc_segment_sum(h2, idxc, zblk)
        u, su = _pass_a(h2, agg, eps[l], W1[l], b1[l])
        v, sv = _pass_b(u, su, g1[l], be1[l], W2[l], b2[l])
        h2 = _pass_c(v, sv, g2[l], be2[l])
    return _pool_cls(h2, batch3, W_cls, b_cls)


# speedup vs baseline: 12.0796x; 1.0042x over previous
import functools

import jax
import jax.numpy as jnp
from jax import lax
from jax.experimental import pallas as pl
from jax.experimental.pallas import tpu as pltpu
from jax.experimental.pallas import tpu_sc as plsc

N = 50000
E = 1600000
H = 64
HH = 32
L = 3
G = 128
C = 2

RB = 2000
NB = N // RB
RBM = 1000
NBM = N // RBM

NPAD = 51200
TRASH = N
TILE_ROWS = NPAD // 16
PTR = 784
EPAD = PTR * 128 * 16
ER = EPAD // 128
CH = 2
NCH = PTR // CH


def _sc_segment_sum(h2, idxc, zblk):
    mesh = plsc.VectorSubcoreMesh(core_axis_name="c", subcore_axis_name="s",
                                  num_cores=2, num_subcores=16)

    @functools.partial(
        pl.kernel,
        out_type=jax.ShapeDtypeStruct((2, NPAD, HH), jnp.float32),
        mesh=mesh,
        compiler_params=pltpu.CompilerParams(use_tc_tiling_on_sc=False),
        scratch_types=[
            pltpu.VMEM((6, 2 * CH, 128), jnp.int32),
            pltpu.VMEM((3, CH, 128, HH), jnp.float32),
            pltpu.VMEM_SHARED((NPAD, HH), jnp.float32),
            pltpu.SemaphoreType.DMA((6,)),
            pltpu.SemaphoreType.DMA((3,)),
            pltpu.SemaphoreType.DMA((3,)),
        ],
    )
    def k(h2_hbm, idx_hbm, z_hbm, out_hbm,
          idx, rows, acc, isem, gsem, ssem):
        c = lax.axis_index("c")
        s = lax.axis_index("s")

        pltpu.sync_copy(z_hbm, rows.at[0, 0])

        def zloop(i, carry):
            pltpu.sync_copy(rows.at[0, 0],
                            acc.at[pl.ds(s * TILE_ROWS + i * 128, 128)])
            return carry
        lax.fori_loop(0, TILE_ROWS // 128, zloop, 0)
        plsc.subcore_barrier()

        def fire_idx(kc):
            q = lax.rem(kc, 6)
            r0 = 2 * (s * PTR + kc * CH)
            pltpu.async_copy(idx_hbm.at[c, pl.ds(r0, 2 * CH)], idx.at[q],
                             isem.at[q])

        def wait_idx(kc):
            q = lax.rem(kc, 6)
            r0 = 2 * (s * PTR + kc * CH)
            pltpu.make_async_copy(idx_hbm.at[c, pl.ds(r0, 2 * CH)], idx.at[q],
                                  isem.at[q]).wait()

        def fire_gathers(kc):
            p = lax.rem(kc, 3)
            q = lax.rem(kc, 6)

            def fj(j, carry):
                pltpu.async_copy(
                    h2_hbm.at[idx.at[q, 2 * j]], rows.at[p, j], gsem.at[p])
                return carry
            lax.fori_loop(0, CH, fj, 0)

        def wait_gathers(kc):
            p = lax.rem(kc, 3)
            q = lax.rem(kc, 6)

            def wj(j, carry):
                pltpu.make_async_copy(
                    h2_hbm.at[idx.at[q, 2 * j]], rows.at[p, j],
                    gsem.at[p]).wait()
                return carry
            lax.fori_loop(0, CH, wj, 0)

        def fire_scatters(kc):
            p = lax.rem(kc, 3)
            q = lax.rem(kc, 6)

            def sj(j, carry):
                pltpu.async_copy(rows.at[p, j], acc.at[idx.at[q, 2 * j + 1]],
                                 ssem.at[p], add=True)
                return carry
            lax.fori_loop(0, CH, sj, 0)

        def wait_scatters(kc):
            p = lax.rem(kc, 3)
            q = lax.rem(kc, 6)

            def wj(j, carry):
                pltpu.make_async_copy(rows.at[p, j], acc.at[idx.at[q, 2 * j + 1]],
                                      ssem.at[p]).wait()
                return carry
            lax.fori_loop(0, CH, wj, 0)

        lax.fori_loop(0, 4, lambda i, cr: (fire_idx(i), cr)[1], 0)

        def prol(i, cr):
            wait_idx(i)
            fire_gathers(i)
            return cr
        lax.fori_loop(0, 2, prol, 0)

        def body(kc, carry):
            @pl.when(kc >= 1)
            def _():
                wait_scatters(kc - 1)

            @pl.when(kc + 2 < NCH)
            def _():
                wait_idx(kc + 2)
                fire_gathers(kc + 2)

            wait_gathers(kc)
            fire_scatters(kc)

            @pl.when(kc + 4 < NCH)
            def _():
                fire_idx(kc + 4)
            return carry
        lax.fori_loop(0, NCH, body, 0)
        wait_scatters(NCH - 1)
        plsc.subcore_barrier()

        def wloop(i, carry):
            b0 = s * TILE_ROWS + i * 128
            pltpu.sync_copy(acc.at[pl.ds(b0, 128)], rows.at[0, 0])
            pltpu.sync_copy(rows.at[0, 0], out_hbm.at[c, pl.ds(b0, 128)])
            return carry
        lax.fori_loop(0, TILE_ROWS // 128, wloop, 0)

    return k(h2, idxc, zblk)


def _encoder(x, W_enc, b_enc):
    def body(x_ref, w_ref, b_ref, h2_ref):
        i = pl.program_id(0)
        hblk = x_ref[...] * w_ref[...] + b_ref[...]
        h2_ref[...] = jnp.where(i < NB, hblk[:, :HH], hblk[:, HH:])

    return pl.pallas_call(
        body,
        grid=(2 * NB,),
        in_specs=[
            pl.BlockSpec((RB, 1), lambda i: (i % NB, 0)),
            pl.BlockSpec((1, H), lambda i: (0, 0)),
            pl.BlockSpec((1, H), lambda i: (0, 0)),
        ],
        out_specs=pl.BlockSpec((RB, HH), lambda i: (i, 0)),
        out_shape=jax.ShapeDtypeStruct((2 * N, HH), jnp.float32),
    )(x, W_enc, b_enc.reshape(1, H))


def _gin_layer(h2, agg, eps_l, W1_l, b1_l, g1_l, be1_l, W2_l, b2_l, g2_l, be2_l):
    def body(hlo_ref, hhi_ref, agg_ref, eps_ref, w1_ref, b1_ref, g1_ref,
             be1_ref, w2_ref, b2_ref, g2_ref, be2_ref, h2_ref,
             u_s, v_s, acc_u, acc_v):
        i = pl.program_id(0)

        @pl.when(i < NBM)
        def _():
            h = jnp.concatenate([hlo_ref[...], hhi_ref[...]], axis=1)
            a = jnp.concatenate([agg_ref[0], agg_ref[1]], axis=1)
            t = eps_ref[0, 0] * h + a
            u = jnp.dot(t, w1_ref[...],
                        preferred_element_type=jnp.float32) + b1_ref[...]
            u_s[pl.ds(i * RBM, RBM), :] = u

            @pl.when(i == 0)
            def _():
                acc_u[...] = jnp.zeros_like(acc_u)
            acc_u[0:1, :] += jnp.sum(u, axis=0, keepdims=True)
            acc_u[1:2, :] += jnp.sum(u * u, axis=0, keepdims=True)

        @pl.when((i >= NBM) & (i < 2 * NBM))
        def _():
            j = i - NBM
            u = u_s[pl.ds(j * RBM, RBM), :]
            mean = acc_u[0:1, :] * (1.0 / N)
            var = acc_u[1:2, :] * (1.0 / N) - mean * mean
            un = (u - mean) * (lax.rsqrt(var + 1e-5) * g1_ref[...]) + be1_ref[...]
            un = jnp.maximum(un, 0.0)
            v = jnp.dot(un, w2_ref[...],
                        preferred_element_type=jnp.float32) + b2_ref[...]
            v_s[pl.ds(j * RBM, RBM), :] = v

            @pl.when(j == 0)
            def _():
                acc_v[...] = jnp.zeros_like(acc_v)
            acc_v[0:1, :] += jnp.sum(v, axis=0, keepdims=True)
            acc_v[1:2, :] += jnp.sum(v * v, axis=0, keepdims=True)

        @pl.when(i >= 2 * NBM)
        def _():
            j = lax.rem(i - 2 * NBM, NBM)
            v = v_s[pl.ds(j * RBM, RBM), :]
            mean = acc_v[0:1, :] * (1.0 / N)
            var = acc_v[1:2, :] * (1.0 / N) - mean * mean
            hn = (v - mean) * (lax.rsqrt(var + 1e-5) * g2_ref[...]) + be2_ref[...]
            hn = jnp.maximum(hn, 0.0)
            h2_ref[...] = jnp.where(i < 3 * NBM, hn[:, :HH], hn[:, HH:])

    def clamp_a(i):
        return jnp.where(i < NBM, i, 0)

    return pl.pallas_call(
        body,
        grid=(4 * NBM,),
        in_specs=[
            pl.BlockSpec((RBM, HH), lambda i: (clamp_a(i), 0)),
            pl.BlockSpec((RBM, HH), lambda i: (NBM + clamp_a(i), 0)),
            pl.BlockSpec((2, RBM, HH), lambda i: (0, clamp_a(i), 0)),
            pl.BlockSpec((1, 1), lambda i: (0, 0)),
            pl.BlockSpec((H, H), lambda i: (0, 0)),
            pl.BlockSpec((1, H), lambda i: (0, 0)),
            pl.BlockSpec((1, H), lambda i: (0, 0)),
            pl.BlockSpec((1, H), lambda i: (0, 0)),
            pl.BlockSpec((H, H), lambda i: (0, 0)),
            pl.BlockSpec((1, H), lambda i: (0, 0)),
            pl.BlockSpec((1, H), lambda i: (0, 0)),
            pl.BlockSpec((1, H), lambda i: (0, 0)),
        ],
        out_specs=pl.BlockSpec(
            (RBM, HH), lambda i: (jnp.where(i >= 2 * NBM, i - 2 * NBM, 0), 0)),
        out_shape=jax.ShapeDtypeStruct((2 * N, HH), jnp.float32),
        scratch_shapes=[
            pltpu.VMEM((N, H), jnp.float32),
            pltpu.VMEM((N, H), jnp.float32),
            pltpu.VMEM((2, H), jnp.float32),
            pltpu.VMEM((2, H), jnp.float32),
        ],
    )(h2, h2, agg, (1.0 + eps_l).reshape(1, 1), W1_l, b1_l.reshape(1, H),
      g1_l.reshape(1, H), be1_l.reshape(1, H), W2_l, b2_l.reshape(1, H),
      g2_l.reshape(1, H), be2_l.reshape(1, H))


def _pool_cls(h2, batch3, W_cls, b_cls):
    def body(lo_ref, hi_ref, b_ref, w_ref, bc_ref, o_ref, acc):
        i = pl.program_id(0)

        @pl.when(i == 0)
        def _():
            acc[...] = jnp.zeros_like(acc)

        h = jnp.concatenate(
            [lo_ref[...], hi_ref[...], jnp.ones((RB, 1), jnp.float32)], axis=1)
        bv = b_ref[0, 0, :]
        onehot = (bv[:, None] ==
                  lax.broadcasted_iota(jnp.int32, (RB, G), 1)).astype(jnp.float32)
        acc[...] += lax.dot_general(
            onehot, h, (((0,), (0,)), ((), ())),
            preferred_element_type=jnp.float32)

        @pl.when(i == NB - 1)
        def _():
            cnt = jnp.maximum(acc[:, H:H + 1], 1.0)
            pooled = acc[:, :H] / cnt
            o_ref[...] = jnp.dot(pooled, w_ref[...],
                                 preferred_element_type=jnp.float32) + bc_ref[...]

    return pl.pallas_call(
        body,
        grid=(NB,),
        in_specs=[
            pl.BlockSpec((RB, HH), lambda i: (i, 0)),
            pl.BlockSpec((RB, HH), lambda i: (NB + i, 0)),
            pl.BlockSpec((1, 1, RB), lambda i: (i, 0, 0)),
            pl.BlockSpec((H, C), lambda i: (0, 0)),
            pl.BlockSpec((1, C), lambda i: (0, 0)),
        ],
        out_specs=pl.BlockSpec((G, C), lambda i: (0, 0)),
        out_shape=jax.ShapeDtypeStruct((G, C), jnp.float32),
        scratch_shapes=[pltpu.VMEM((G, H + 1), jnp.float32)],
    )(h2, h2, batch3, W_cls, b_cls.reshape(1, C))


def kernel(x, edge_index, batch, W_enc, b_enc, eps, W1, b1, g1, be1,
           W2, b2, g2, be2, W_cls, b_cls):
    src = edge_index[0].astype(jnp.int32)
    dst = edge_index[1].astype(jnp.int32)
    padn = EPAD - E
    src2 = jnp.concatenate([src, jnp.zeros((padn,), jnp.int32)]).reshape(ER, 128)
    dst2 = jnp.concatenate([dst, jnp.full((padn,), TRASH, jnp.int32)]).reshape(ER, 128)
    idxc = jnp.stack(
        [jnp.stack([src2, dst2], axis=1).reshape(2 * ER, 128),
         jnp.stack([src2 + N, dst2], axis=1).reshape(2 * ER, 128)])
    zblk = jnp.zeros((128, HH), jnp.float32)
    batch3 = batch.astype(jnp.int32).reshape(NB, 1, RB)

    h2 = _encoder(x, W_enc, b_enc)
    for l in range(L):
        agg = _sc_segment_sum(h2, idxc, zblk)
        h2 = _gin_layer(h2, agg, eps[l], W1[l], b1[l], g1[l], be1[l],
                        W2[l], b2[l], g2[l], be2[l])
    return _pool_cls(h2, batch3, W_cls, b_cls)


# speedup vs baseline: 12.9477x; 1.0719x over previous
import functools

import jax
import jax.numpy as jnp
from jax import lax
from jax.experimental import pallas as pl
from jax.experimental.pallas import tpu as pltpu
from jax.experimental.pallas import tpu_sc as plsc

N = 50000
E = 1600000
H = 64
HH = 32
L = 3
G = 128
C = 2

RB = 2000
NB = N // RB
RBM = 1000
NBM = N // RBM

NPAD = 51200
TRASH = N
TILE_ROWS = NPAD // 16
PTR = 784
EPAD = PTR * 128 * 16
ER = EPAD // 128
CH = 2
NCH = PTR // CH


def _sc_segment_sum(h_lo, h_hi, idxc, zblk):
    mesh = plsc.VectorSubcoreMesh(core_axis_name="c", subcore_axis_name="s",
                                  num_cores=2, num_subcores=16)

    @functools.partial(
        pl.kernel,
        out_type=jax.ShapeDtypeStruct((2, NPAD, HH), jnp.float32),
        mesh=mesh,
        compiler_params=pltpu.CompilerParams(use_tc_tiling_on_sc=False),
        scratch_types=[
            pltpu.VMEM((6, 2 * CH, 128), jnp.int32),
            pltpu.VMEM((3, CH, 128, HH), jnp.float32),
            pltpu.VMEM_SHARED((NPAD, HH), jnp.float32),
            pltpu.SemaphoreType.DMA((6,)),
            pltpu.SemaphoreType.DMA((3,)),
            pltpu.SemaphoreType.DMA((3,)),
        ],
    )
    def k(hlo_hbm, hhi_hbm, idx_hbm, z_hbm, out_hbm,
          idx, rows, acc, isem, gsem, ssem):
        c = lax.axis_index("c")
        s = lax.axis_index("s")

        pltpu.sync_copy(z_hbm, rows.at[0, 0])

        def zloop(i, carry):
            pltpu.sync_copy(rows.at[0, 0],
                            acc.at[pl.ds(s * TILE_ROWS + i * 128, 128)])
            return carry
        lax.fori_loop(0, TILE_ROWS // 128, zloop, 0)
        plsc.subcore_barrier()

        def fire_idx(kc):
            q = lax.rem(kc, 6)
            r0 = 2 * (s * PTR + kc * CH)
            pltpu.async_copy(idx_hbm.at[pl.ds(r0, 2 * CH)], idx.at[q],
                             isem.at[q])

        def wait_idx(kc):
            q = lax.rem(kc, 6)
            r0 = 2 * (s * PTR + kc * CH)
            pltpu.make_async_copy(idx_hbm.at[pl.ds(r0, 2 * CH)], idx.at[q],
                                  isem.at[q]).wait()

        def fire_gathers(kc):
            p = lax.rem(kc, 3)
            q = lax.rem(kc, 6)

            def fj(j, carry):
                d_lo = pltpu.make_async_copy(
                    hlo_hbm.at[idx.at[q, 2 * j]], rows.at[p, j], gsem.at[p])
                d_hi = pltpu.make_async_copy(
                    hhi_hbm.at[idx.at[q, 2 * j]], rows.at[p, j], gsem.at[p])

                @pl.when(c == 0)
                def _():
                    d_lo.start()

                @pl.when(c == 1)
                def _():
                    d_hi.start()
                return carry
            lax.fori_loop(0, CH, fj, 0)

        def wait_gathers(kc):
            p = lax.rem(kc, 3)
            q = lax.rem(kc, 6)

            def wj(j, carry):
                pltpu.make_async_copy(
                    hlo_hbm.at[idx.at[q, 2 * j]], rows.at[p, j],
                    gsem.at[p]).wait()
                return carry
            lax.fori_loop(0, CH, wj, 0)

        def fire_scatters(kc):
            p = lax.rem(kc, 3)
            q = lax.rem(kc, 6)

            def sj(j, carry):
                pltpu.async_copy(rows.at[p, j], acc.at[idx.at[q, 2 * j + 1]],
                                 ssem.at[p], add=True)
                return carry
            lax.fori_loop(0, CH, sj, 0)

        def wait_scatters(kc):
            p = lax.rem(kc, 3)
            q = lax.rem(kc, 6)

            def wj(j, carry):
                pltpu.make_async_copy(rows.at[p, j], acc.at[idx.at[q, 2 * j + 1]],
                                      ssem.at[p]).wait()
                return carry
            lax.fori_loop(0, CH, wj, 0)

        lax.fori_loop(0, 4, lambda i, cr: (fire_idx(i), cr)[1], 0)

        def prol(i, cr):
            wait_idx(i)
            fire_gathers(i)
            return cr
        lax.fori_loop(0, 2, prol, 0)

        def body(kc, carry):
            @pl.when(kc >= 1)
            def _():
                wait_scatters(kc - 1)

            @pl.when(kc + 2 < NCH)
            def _():
                wait_idx(kc + 2)
                fire_gathers(kc + 2)

            wait_gathers(kc)
            fire_scatters(kc)

            @pl.when(kc + 4 < NCH)
            def _():
                fire_idx(kc + 4)
            return carry
        lax.fori_loop(0, NCH, body, 0)
        wait_scatters(NCH - 1)
        plsc.subcore_barrier()

        def wloop(i, carry):
            b0 = s * TILE_ROWS + i * 128
            pltpu.sync_copy(acc.at[pl.ds(b0, 128)], rows.at[0, 0])
            pltpu.sync_copy(rows.at[0, 0], out_hbm.at[c, pl.ds(b0, 128)])
            return carry
        lax.fori_loop(0, TILE_ROWS // 128, wloop, 0)

    return k(h_lo, h_hi, idxc, zblk)


def _encoder(x, W_enc, b_enc):
    def body(x_ref, w_ref, b_ref, lo_ref, hi_ref):
        hblk = x_ref[...] * w_ref[...] + b_ref[...]
        lo_ref[...] = hblk[:, :HH]
        hi_ref[...] = hblk[:, HH:]

    return pl.pallas_call(
        body,
        grid=(NB,),
        in_specs=[
            pl.BlockSpec((RB, 1), lambda i: (i, 0)),
            pl.BlockSpec((1, H), lambda i: (0, 0)),
            pl.BlockSpec((1, H), lambda i: (0, 0)),
        ],
        out_specs=[pl.BlockSpec((RB, HH), lambda i: (i, 0))] * 2,
        out_shape=[jax.ShapeDtypeStruct((N, HH), jnp.float32)] * 2,
    )(x, W_enc, b_enc.reshape(1, H))


def _gin_layer(h_lo, h_hi, agg, eps_l, W1_l, b1_l, g1_l, be1_l,
               W2_l, b2_l, g2_l, be2_l):
    def body(hlo_ref, hhi_ref, agg_ref, eps_ref, w1_ref, b1_ref, g1_ref,
             be1_ref, w2_ref, b2_ref, g2_ref, be2_ref, lo_ref, hi_ref,
             v_s, acc_u, acc_v):
        i = pl.program_id(0)

        def compute_u():
            h = jnp.concatenate([hlo_ref[...], hhi_ref[...]], axis=1)
            a = jnp.concatenate([agg_ref[0], agg_ref[1]], axis=1)
            t = eps_ref[0, 0] * h + a
            return jnp.dot(t, w1_ref[...],
                           preferred_element_type=jnp.float32) + b1_ref[...]

        @pl.when(i < NB)
        def _():
            u = compute_u()

            @pl.when(i == 0)
            def _():
                acc_u[...] = jnp.zeros_like(acc_u)
            acc_u[0:1, :] += jnp.sum(u, axis=0, keepdims=True)
            acc_u[1:2, :] += jnp.sum(u * u, axis=0, keepdims=True)

        @pl.when((i >= NB) & (i < 2 * NB))
        def _():
            j = i - NB
            u = compute_u()
            mean = acc_u[0:1, :] * (1.0 / N)
            var = acc_u[1:2, :] * (1.0 / N) - mean * mean
            un = (u - mean) * (lax.rsqrt(var + 1e-5) * g1_ref[...]) + be1_ref[...]
            un = jnp.maximum(un, 0.0)
            v = jnp.dot(un, w2_ref[...],
                        preferred_element_type=jnp.float32) + b2_ref[...]
            v_s[pl.ds(j * RB, RB), :] = v

            @pl.when(j == 0)
            def _():
                acc_v[...] = jnp.zeros_like(acc_v)
            acc_v[0:1, :] += jnp.sum(v, axis=0, keepdims=True)
            acc_v[1:2, :] += jnp.sum(v * v, axis=0, keepdims=True)

        @pl.when(i >= 2 * NB)
        def _():
            j = i - 2 * NB
            v = v_s[pl.ds(j * RB, RB), :]
            mean = acc_v[0:1, :] * (1.0 / N)
            var = acc_v[1:2, :] * (1.0 / N) - mean * mean
            hn = (v - mean) * (lax.rsqrt(var + 1e-5) * g2_ref[...]) + be2_ref[...]
            hn = jnp.maximum(hn, 0.0)
            lo_ref[...] = hn[:, :HH]
            hi_ref[...] = hn[:, HH:]

    def clamp_ab(i):
        return jnp.where(i < 2 * NB, lax.rem(i, NB), 0)

    def clamp_c(i):
        return jnp.where(i >= 2 * NB, i - 2 * NB, 0)

    return pl.pallas_call(
        body,
        grid=(3 * NB,),
        in_specs=[
            pl.BlockSpec((RB, HH), lambda i: (clamp_ab(i), 0)),
            pl.BlockSpec((RB, HH), lambda i: (clamp_ab(i), 0)),
            pl.BlockSpec((2, RB, HH), lambda i: (0, clamp_ab(i), 0)),
            pl.BlockSpec((1, 1), lambda i: (0, 0)),
            pl.BlockSpec((H, H), lambda i: (0, 0)),
            pl.BlockSpec((1, H), lambda i: (0, 0)),
            pl.BlockSpec((1, H), lambda i: (0, 0)),
            pl.BlockSpec((1, H), lambda i: (0, 0)),
            pl.BlockSpec((H, H), lambda i: (0, 0)),
            pl.BlockSpec((1, H), lambda i: (0, 0)),
            pl.BlockSpec((1, H), lambda i: (0, 0)),
            pl.BlockSpec((1, H), lambda i: (0, 0)),
        ],
        out_specs=[pl.BlockSpec((RB, HH), lambda i: (clamp_c(i), 0))] * 2,
        out_shape=[jax.ShapeDtypeStruct((N, HH), jnp.float32)] * 2,
        scratch_shapes=[
            pltpu.VMEM((N, H), jnp.float32),
            pltpu.VMEM((2, H), jnp.float32),
            pltpu.VMEM((2, H), jnp.float32),
        ],
    )(h_lo, h_hi, agg, (1.0 + eps_l).reshape(1, 1), W1_l, b1_l.reshape(1, H),
      g1_l.reshape(1, H), be1_l.reshape(1, H), W2_l, b2_l.reshape(1, H),
      g2_l.reshape(1, H), be2_l.reshape(1, H))


def _pool_cls(h_lo, h_hi, batch3, W_cls, b_cls):
    def body(lo_ref, hi_ref, b_ref, w_ref, bc_ref, o_ref, acc):
        i = pl.program_id(0)

        @pl.when(i == 0)
        def _():
            acc[...] = jnp.zeros_like(acc)

        h = jnp.concatenate(
            [lo_ref[...], hi_ref[...], jnp.ones((RB, 1), jnp.float32)], axis=1)
        bv = b_ref[0, 0, :]
        onehot = (bv[:, None] ==
                  lax.broadcasted_iota(jnp.int32, (RB, G), 1)).astype(jnp.float32)
        acc[...] += lax.dot_general(
            onehot, h, (((0,), (0,)), ((), ())),
            preferred_element_type=jnp.float32)

        @pl.when(i == NB - 1)
        def _():
            cnt = jnp.maximum(acc[:, H:H + 1], 1.0)
            pooled = acc[:, :H] / cnt
            o_ref[...] = jnp.dot(pooled, w_ref[...],
                                 preferred_element_type=jnp.float32) + bc_ref[...]

    return pl.pallas_call(
        body,
        grid=(NB,),
        in_specs=[
            pl.BlockSpec((RB, HH), lambda i: (i, 0)),
            pl.BlockSpec((RB, HH), lambda i: (i, 0)),
            pl.BlockSpec((1, 1, RB), lambda i: (i, 0, 0)),
            pl.BlockSpec((H, C), lambda i: (0, 0)),
            pl.BlockSpec((1, C), lambda i: (0, 0)),
        ],
        out_specs=pl.BlockSpec((G, C), lambda i: (0, 0)),
        out_shape=jax.ShapeDtypeStruct((G, C), jnp.float32),
        scratch_shapes=[pltpu.VMEM((G, H + 1), jnp.float32)],
    )(h_lo, h_hi, batch3, W_cls, b_cls.reshape(1, C))


def kernel(x, edge_index, batch, W_enc, b_enc, eps, W1, b1, g1, be1,
           W2, b2, g2, be2, W_cls, b_cls):
    src = edge_index[0].astype(jnp.int32)
    dst = edge_index[1].astype(jnp.int32)
    padn = EPAD - E
    src2 = jnp.concatenate([src, jnp.zeros((padn,), jnp.int32)]).reshape(ER, 128)
    dst2 = jnp.concatenate([dst, jnp.full((padn,), TRASH, jnp.int32)]).reshape(ER, 128)
    idxc = jnp.stack([src2, dst2], axis=1).reshape(2 * ER, 128)
    zblk = jnp.zeros((128, HH), jnp.float32)
    batch3 = batch.astype(jnp.int32).reshape(NB, 1, RB)

    h_lo, h_hi = _encoder(x, W_enc, b_enc)
    for l in range(L):
        agg = _sc_segment_sum(h_lo, h_hi, idxc, zblk)
        h_lo, h_hi = _gin_layer(h_lo, h_hi, agg, eps[l], W1[l], b1[l],
                                g1[l], be1[l], W2[l], b2[l], g2[l], be2[l])
    return _pool_cls(h_lo, h_hi, batch3, W_cls, b_cls)


# speedup vs baseline: 13.5253x; 1.0446x over previous
import functools

import jax
import jax.numpy as jnp
from jax import lax
from jax.experimental import pallas as pl
from jax.experimental.pallas import tpu as pltpu
from jax.experimental.pallas import tpu_sc as plsc

N = 50000
E = 1600000
H = 64
HH = 32
L = 3
G = 128
C = 2

RB = 2000
NB = N // RB
RBM = 5000
NBM = N // RBM

NPAD = 51200
TRASH = N
TILE_ROWS = NPAD // 16
PTR = 784
EPAD = PTR * 128 * 16
ER = EPAD // 128
CH = 2
NCH = PTR // CH


def _sc_segment_sum(h_lo, h_hi, idxc, zblk):
    mesh = plsc.VectorSubcoreMesh(core_axis_name="c", subcore_axis_name="s",
                                  num_cores=2, num_subcores=16)

    @functools.partial(
        pl.kernel,
        out_type=jax.ShapeDtypeStruct((2, NPAD, HH), jnp.float32),
        mesh=mesh,
        compiler_params=pltpu.CompilerParams(use_tc_tiling_on_sc=False),
        scratch_types=[
            pltpu.VMEM((6, 2 * CH, 128), jnp.int32),
            pltpu.VMEM((3, CH, 128, HH), jnp.float32),
            pltpu.VMEM_SHARED((NPAD, HH), jnp.float32),
            pltpu.SemaphoreType.DMA((6,)),
            pltpu.SemaphoreType.DMA((3,)),
            pltpu.SemaphoreType.DMA((3,)),
        ],
    )
    def k(hlo_hbm, hhi_hbm, idx_hbm, z_hbm, out_hbm,
          idx, rows, acc, isem, gsem, ssem):
        c = lax.axis_index("c")
        s = lax.axis_index("s")

        pltpu.sync_copy(z_hbm, rows.at[0, 0])

        def zloop(i, carry):
            pltpu.sync_copy(rows.at[0, 0],
                            acc.at[pl.ds(s * TILE_ROWS + i * 128, 128)])
            return carry
        lax.fori_loop(0, TILE_ROWS // 128, zloop, 0)
        plsc.subcore_barrier()

        def fire_idx(kc):
            q = lax.rem(kc, 6)
            r0 = 2 * (s * PTR + kc * CH)
            pltpu.async_copy(idx_hbm.at[pl.ds(r0, 2 * CH)], idx.at[q],
                             isem.at[q])

        def wait_idx(kc):
            q = lax.rem(kc, 6)
            r0 = 2 * (s * PTR + kc * CH)
            pltpu.make_async_copy(idx_hbm.at[pl.ds(r0, 2 * CH)], idx.at[q],
                                  isem.at[q]).wait()

        def fire_gathers(kc):
            p = lax.rem(kc, 3)
            q = lax.rem(kc, 6)

            def fj(j, carry):
                d_lo = pltpu.make_async_copy(
                    hlo_hbm.at[idx.at[q, 2 * j]], rows.at[p, j], gsem.at[p])
                d_hi = pltpu.make_async_copy(
                    hhi_hbm.at[idx.at[q, 2 * j]], rows.at[p, j], gsem.at[p])

                @pl.when(c == 0)
                def _():
                    d_lo.start()

                @pl.when(c == 1)
                def _():
                    d_hi.start()
                return carry
            lax.fori_loop(0, CH, fj, 0)

        def wait_gathers(kc):
            p = lax.rem(kc, 3)
            q = lax.rem(kc, 6)

            def wj(j, carry):
                pltpu.make_async_copy(
                    hlo_hbm.at[idx.at[q, 2 * j]], rows.at[p, j],
                    gsem.at[p]).wait()
                return carry
            lax.fori_loop(0, CH, wj, 0)

        def fire_scatters(kc):
            p = lax.rem(kc, 3)
            q = lax.rem(kc, 6)

            def sj(j, carry):
                pltpu.async_copy(rows.at[p, j], acc.at[idx.at[q, 2 * j + 1]],
                                 ssem.at[p], add=True)
                return carry
            lax.fori_loop(0, CH, sj, 0)

        def wait_scatters(kc):
            p = lax.rem(kc, 3)
            q = lax.rem(kc, 6)

            def wj(j, carry):
                pltpu.make_async_copy(rows.at[p, j], acc.at[idx.at[q, 2 * j + 1]],
                                      ssem.at[p]).wait()
                return carry
            lax.fori_loop(0, CH, wj, 0)

        lax.fori_loop(0, 4, lambda i, cr: (fire_idx(i), cr)[1], 0)

        def prol(i, cr):
            wait_idx(i)
            fire_gathers(i)
            return cr
        lax.fori_loop(0, 2, prol, 0)

        def body(kc, carry):
            @pl.when(kc >= 1)
            def _():
                wait_scatters(kc - 1)

            @pl.when(kc + 2 < NCH)
            def _():
                wait_idx(kc + 2)
                fire_gathers(kc + 2)

            wait_gathers(kc)
            fire_scatters(kc)

            @pl.when(kc + 4 < NCH)
            def _():
                fire_idx(kc + 4)
            return carry
        lax.fori_loop(0, NCH, body, 0)
        wait_scatters(NCH - 1)
        plsc.subcore_barrier()

        def wloop(i, carry):
            b0 = s * TILE_ROWS + i * 128
            pltpu.sync_copy(acc.at[pl.ds(b0, 128)], rows.at[0, 0])
            pltpu.sync_copy(rows.at[0, 0], out_hbm.at[c, pl.ds(b0, 128)])
            return carry
        lax.fori_loop(0, TILE_ROWS // 128, wloop, 0)

    return k(h_lo, h_hi, idxc, zblk)


def _encoder(x, W_enc, b_enc):
    def body(x_ref, w_ref, b_ref, lo_ref, hi_ref):
        hblk = x_ref[...] * w_ref[...] + b_ref[...]
        lo_ref[...] = hblk[:, :HH]
        hi_ref[...] = hblk[:, HH:]

    return pl.pallas_call(
        body,
        grid=(NB,),
        in_specs=[
            pl.BlockSpec((RB, 1), lambda i: (i, 0)),
            pl.BlockSpec((1, H), lambda i: (0, 0)),
            pl.BlockSpec((1, H), lambda i: (0, 0)),
        ],
        out_specs=[pl.BlockSpec((RB, HH), lambda i: (i, 0))] * 2,
        out_shape=[jax.ShapeDtypeStruct((N, HH), jnp.float32)] * 2,
    )(x, W_enc, b_enc.reshape(1, H))


def _gin_layer(h_lo, h_hi, agg, eps_l, W1_l, b1_l, g1_l, be1_l,
               W2_l, b2_l, g2_l, be2_l):
    def body(hlo_ref, hhi_ref, agg_ref, eps_ref, w1_ref, b1_ref, g1_ref,
             be1_ref, w2_ref, b2_ref, g2_ref, be2_ref, lo_ref, hi_ref,
             v_s, acc_u, acc_v):
        i = pl.program_id(0)

        def compute_u():
            h = jnp.concatenate([hlo_ref[...], hhi_ref[...]], axis=1)
            a = jnp.concatenate([agg_ref[0], agg_ref[1]], axis=1)
            t = eps_ref[0, 0] * h + a
            return jnp.dot(t, w1_ref[...],
                           preferred_element_type=jnp.float32) + b1_ref[...]

        @pl.when(i < NBM)
        def _():
            u = compute_u()

            @pl.when(i == 0)
            def _():
                acc_u[...] = jnp.zeros_like(acc_u)
            acc_u[0:1, :] += jnp.sum(u, axis=0, keepdims=True)
            acc_u[1:2, :] += jnp.sum(u * u, axis=0, keepdims=True)

        @pl.when((i >= NBM) & (i < 2 * NBM))
        def _():
            j = i - NBM
            u = compute_u()
            mean = acc_u[0:1, :] * (1.0 / N)
            var = acc_u[1:2, :] * (1.0 / N) - mean * mean
            un = (u - mean) * (lax.rsqrt(var + 1e-5) * g1_ref[...]) + be1_ref[...]
            un = jnp.maximum(un, 0.0)
            v = jnp.dot(un, w2_ref[...],
                        preferred_element_type=jnp.float32) + b2_ref[...]
            v_s[pl.ds(j * RBM, RBM), :] = v

            @pl.when(j == 0)
            def _():
                acc_v[...] = jnp.zeros_like(acc_v)
            acc_v[0:1, :] += jnp.sum(v, axis=0, keepdims=True)
            acc_v[1:2, :] += jnp.sum(v * v, axis=0, keepdims=True)

        @pl.when(i >= 2 * NBM)
        def _():
            j = i - 2 * NBM
            v = v_s[pl.ds(j * RBM, RBM), :]
            mean = acc_v[0:1, :] * (1.0 / N)
            var = acc_v[1:2, :] * (1.0 / N) - mean * mean
            hn = (v - mean) * (lax.rsqrt(var + 1e-5) * g2_ref[...]) + be2_ref[...]
            hn = jnp.maximum(hn, 0.0)
            lo_ref[...] = hn[:, :HH]
            hi_ref[...] = hn[:, HH:]

    def clamp_ab(i):
        return jnp.where(i < 2 * NBM, lax.rem(i, NBM), 0)

    def clamp_c(i):
        return jnp.where(i >= 2 * NBM, i - 2 * NBM, 0)

    return pl.pallas_call(
        body,
        grid=(3 * NBM,),
        in_specs=[
            pl.BlockSpec((RBM, HH), lambda i: (clamp_ab(i), 0)),
            pl.BlockSpec((RBM, HH), lambda i: (clamp_ab(i), 0)),
            pl.BlockSpec((2, RBM, HH), lambda i: (0, clamp_ab(i), 0)),
            pl.BlockSpec((1, 1), lambda i: (0, 0)),
            pl.BlockSpec((H, H), lambda i: (0, 0)),
            pl.BlockSpec((1, H), lambda i: (0, 0)),
            pl.BlockSpec((1, H), lambda i: (0, 0)),
            pl.BlockSpec((1, H), lambda i: (0, 0)),
            pl.BlockSpec((H, H), lambda i: (0, 0)),
            pl.BlockSpec((1, H), lambda i: (0, 0)),
            pl.BlockSpec((1, H), lambda i: (0, 0)),
            pl.BlockSpec((1, H), lambda i: (0, 0)),
        ],
        out_specs=[pl.BlockSpec((RBM, HH), lambda i: (clamp_c(i), 0))] * 2,
        out_shape=[jax.ShapeDtypeStruct((N, HH), jnp.float32)] * 2,
        scratch_shapes=[
            pltpu.VMEM((N, H), jnp.float32),
            pltpu.VMEM((2, H), jnp.float32),
            pltpu.VMEM((2, H), jnp.float32),
        ],
    )(h_lo, h_hi, agg, (1.0 + eps_l).reshape(1, 1), W1_l, b1_l.reshape(1, H),
      g1_l.reshape(1, H), be1_l.reshape(1, H), W2_l, b2_l.reshape(1, H),
      g2_l.reshape(1, H), be2_l.reshape(1, H))


def _pool_cls(h_lo, h_hi, batch3, W_cls, b_cls):
    def body(lo_ref, hi_ref, b_ref, w_ref, bc_ref, o_ref, acc):
        i = pl.program_id(0)

        @pl.when(i == 0)
        def _():
            acc[...] = jnp.zeros_like(acc)

        h = jnp.concatenate(
            [lo_ref[...], hi_ref[...], jnp.ones((RB, 1), jnp.float32)], axis=1)
        bv = b_ref[0, 0, :]
        onehot = (bv[:, None] ==
                  lax.broadcasted_iota(jnp.int32, (RB, G), 1)).astype(jnp.float32)
        acc[...] += lax.dot_general(
            onehot, h, (((0,), (0,)), ((), ())),
            preferred_element_type=jnp.float32)

        @pl.when(i == NB - 1)
        def _():
            cnt = jnp.maximum(acc[:, H:H + 1], 1.0)
            pooled = acc[:, :H] / cnt
            o_ref[...] = jnp.dot(pooled, w_ref[...],
                                 preferred_element_type=jnp.float32) + bc_ref[...]

    return pl.pallas_call(
        body,
        grid=(NB,),
        in_specs=[
            pl.BlockSpec((RB, HH), lambda i: (i, 0)),
            pl.BlockSpec((RB, HH), lambda i: (i, 0)),
            pl.BlockSpec((1, 1, RB), lambda i: (i, 0, 0)),
            pl.BlockSpec((H, C), lambda i: (0, 0)),
            pl.BlockSpec((1, C), lambda i: (0, 0)),
        ],
        out_specs=pl.BlockSpec((G, C), lambda i: (0, 0)),
        out_shape=jax.ShapeDtypeStruct((G, C), jnp.float32),
        scratch_shapes=[pltpu.VMEM((G, H + 1), jnp.float32)],
    )(h_lo, h_hi, batch3, W_cls, b_cls.reshape(1, C))


def kernel(x, edge_index, batch, W_enc, b_enc, eps, W1, b1, g1, be1,
           W2, b2, g2, be2, W_cls, b_cls):
    src = edge_index[0].astype(jnp.int32)
    dst = edge_index[1].astype(jnp.int32)
    padn = EPAD - E
    src2 = jnp.concatenate([src, jnp.zeros((padn,), jnp.int32)]).reshape(ER, 128)
    dst2 = jnp.concatenate([dst, jnp.full((padn,), TRASH, jnp.int32)]).reshape(ER, 128)
    idxc = jnp.stack([src2, dst2], axis=1).reshape(2 * ER, 128)
    zblk = jnp.zeros((128, HH), jnp.float32)
    batch3 = batch.astype(jnp.int32).reshape(NB, 1, RB)

    h_lo, h_hi = _encoder(x, W_enc, b_enc)
    for l in range(L):
        agg = _sc_segment_sum(h_lo, h_hi, idxc, zblk)
        h_lo, h_hi = _gin_layer(h_lo, h_hi, agg, eps[l], W1[l], b1[l],
                                g1[l], be1[l], W2[l], b2[l], g2[l], be2[l])
    return _pool_cls(h_lo, h_hi, batch3, W_cls, b_cls)


# speedup vs baseline: 13.7081x; 1.0135x over previous
import functools

import jax
import jax.numpy as jnp
from jax import lax
from jax.experimental import pallas as pl
from jax.experimental.pallas import tpu as pltpu
from jax.experimental.pallas import tpu_sc as plsc

N = 50000
E = 1600000
H = 64
HH = 32
L = 3
G = 128
C = 2

RB = 2000
NB = N // RB
RBM = 5000
NBM = N // RBM

NPAD = 51200
TRASH = N
TILE_ROWS = NPAD // 16
PTR = 784
EPAD = PTR * 128 * 16
ER = EPAD // 128
CH = 2
NCH = PTR // CH


def _sc_segment_sum(h_lo, h_hi, idxc, zblk):
    mesh = plsc.VectorSubcoreMesh(core_axis_name="c", subcore_axis_name="s",
                                  num_cores=2, num_subcores=16)

    @functools.partial(
        pl.kernel,
        out_type=jax.ShapeDtypeStruct((2, NPAD, HH), jnp.float32),
        mesh=mesh,
        compiler_params=pltpu.CompilerParams(use_tc_tiling_on_sc=False),
        scratch_types=[
            pltpu.VMEM((6, 2 * CH, 128), jnp.int32),
            pltpu.VMEM((3, CH, 128, HH), jnp.float32),
            pltpu.VMEM_SHARED((NPAD, HH), jnp.float32),
            pltpu.SemaphoreType.DMA((6,)),
            pltpu.SemaphoreType.DMA((3,)),
            pltpu.SemaphoreType.DMA((3,)),
        ],
    )
    def k(hlo_hbm, hhi_hbm, idx_hbm, z_hbm, out_hbm,
          idx, rows, acc, isem, gsem, ssem):
        c = lax.axis_index("c")
        s = lax.axis_index("s")

        pltpu.sync_copy(z_hbm, rows.at[0, 0])

        def zloop(i, carry):
            pltpu.async_copy(rows.at[0, 0],
                             acc.at[pl.ds(s * TILE_ROWS + i * 128, 128)],
                             gsem.at[0])
            return carry
        lax.fori_loop(0, TILE_ROWS // 128, zloop, 0)

        def zdrain(i, carry):
            pltpu.make_async_copy(
                rows.at[0, 0], acc.at[pl.ds(s * TILE_ROWS + i * 128, 128)],
                gsem.at[0]).wait()
            return carry
        lax.fori_loop(0, TILE_ROWS // 128, zdrain, 0)
        plsc.subcore_barrier()

        def fire_idx(kc):
            q = lax.rem(kc, 6)
            r0 = 2 * (s * PTR + kc * CH)
            pltpu.async_copy(idx_hbm.at[pl.ds(r0, 2 * CH)], idx.at[q],
                             isem.at[q])

        def wait_idx(kc):
            q = lax.rem(kc, 6)
            r0 = 2 * (s * PTR + kc * CH)
            pltpu.make_async_copy(idx_hbm.at[pl.ds(r0, 2 * CH)], idx.at[q],
                                  isem.at[q]).wait()

        def fire_gathers(kc):
            p = lax.rem(kc, 3)
            q = lax.rem(kc, 6)

            def fj(j, carry):
                d_lo = pltpu.make_async_copy(
                    hlo_hbm.at[idx.at[q, 2 * j]], rows.at[p, j], gsem.at[p])
                d_hi = pltpu.make_async_copy(
                    hhi_hbm.at[idx.at[q, 2 * j]], rows.at[p, j], gsem.at[p])

                @pl.when(c == 0)
                def _():
                    d_lo.start()

                @pl.when(c == 1)
                def _():
                    d_hi.start()
                return carry
            lax.fori_loop(0, CH, fj, 0)

        def wait_gathers(kc):
            p = lax.rem(kc, 3)
            q = lax.rem(kc, 6)

            def wj(j, carry):
                pltpu.make_async_copy(
                    hlo_hbm.at[idx.at[q, 2 * j]], rows.at[p, j],
                    gsem.at[p]).wait()
                return carry
            lax.fori_loop(0, CH, wj, 0)

        def fire_scatters(kc):
            p = lax.rem(kc, 3)
            q = lax.rem(kc, 6)

            def sj(j, carry):
                pltpu.async_copy(rows.at[p, j], acc.at[idx.at[q, 2 * j + 1]],
                                 ssem.at[p], add=True)
                return carry
            lax.fori_loop(0, CH, sj, 0)

        def wait_scatters(kc):
            p = lax.rem(kc, 3)
            q = lax.rem(kc, 6)

            def wj(j, carry):
                pltpu.make_async_copy(rows.at[p, j], acc.at[idx.at[q, 2 * j + 1]],
                                      ssem.at[p]).wait()
                return carry
            lax.fori_loop(0, CH, wj, 0)

        lax.fori_loop(0, 4, lambda i, cr: (fire_idx(i), cr)[1], 0)

        def prol(i, cr):
            wait_idx(i)
            fire_gathers(i)
            return cr
        lax.fori_loop(0, 2, prol, 0)

        def body(kc, carry):
            @pl.when(kc >= 1)
            def _():
                wait_scatters(kc - 1)

            @pl.when(kc + 2 < NCH)
            def _():
                wait_idx(kc + 2)
                fire_gathers(kc + 2)

            wait_gathers(kc)
            fire_scatters(kc)

            @pl.when(kc + 4 < NCH)
            def _():
                fire_idx(kc + 4)
            return carry
        lax.fori_loop(0, NCH, body, 0)
        wait_scatters(NCH - 1)
        plsc.subcore_barrier()

        NWB = TILE_ROWS // 128

        def rd(i):
            p = lax.rem(i, 3)
            pltpu.async_copy(acc.at[pl.ds(s * TILE_ROWS + i * 128, 128)],
                             rows.at[p, 0], gsem.at[p])

        def rd_wait(i):
            p = lax.rem(i, 3)
            pltpu.make_async_copy(
                acc.at[pl.ds(s * TILE_ROWS + i * 128, 128)], rows.at[p, 0],
                gsem.at[p]).wait()

        def wr(i):
            p = lax.rem(i, 3)
            pltpu.async_copy(rows.at[p, 0],
                             out_hbm.at[c, pl.ds(s * TILE_ROWS + i * 128, 128)],
                             ssem.at[p])

        def wr_wait(i):
            p = lax.rem(i, 3)
            pltpu.make_async_copy(
                rows.at[p, 0],
                out_hbm.at[c, pl.ds(s * TILE_ROWS + i * 128, 128)],
                ssem.at[p]).wait()

        rd(0)
        rd(1)

        def wloop(i, carry):
            @pl.when(i >= 1)
            def _():
                wr_wait(i - 1)

            rd_wait(i)
            wr(i)

            @pl.when(i + 2 < NWB)
            def _():
                rd(i + 2)
            return carry
        lax.fori_loop(0, NWB, wloop, 0)
        wr_wait(NWB - 1)

    return k(h_lo, h_hi, idxc, zblk)


def _encoder(x, W_enc, b_enc):
    def body(x_ref, w_ref, b_ref, lo_ref, hi_ref):
        hblk = x_ref[...] * w_ref[...] + b_ref[...]
        lo_ref[...] = hblk[:, :HH]
        hi_ref[...] = hblk[:, HH:]

    return pl.pallas_call(
        body,
        grid=(NB,),
        in_specs=[
            pl.BlockSpec((RB, 1), lambda i: (i, 0)),
            pl.BlockSpec((1, H), lambda i: (0, 0)),
            pl.BlockSpec((1, H), lambda i: (0, 0)),
        ],
        out_specs=[pl.BlockSpec((RB, HH), lambda i: (i, 0))] * 2,
        out_shape=[jax.ShapeDtypeStruct((N, HH), jnp.float32)] * 2,
    )(x, W_enc, b_enc.reshape(1, H))


def _gin_layer(h_lo, h_hi, agg, eps_l, W1_l, b1_l, g1_l, be1_l,
               W2_l, b2_l, g2_l, be2_l):
    def body(hlo_ref, hhi_ref, agg_ref, eps_ref, w1_ref, b1_ref, g1_ref,
             be1_ref, w2_ref, b2_ref, g2_ref, be2_ref, lo_ref, hi_ref,
             v_s, acc_u, acc_v):
        i = pl.program_id(0)

        def compute_u():
            h = jnp.concatenate([hlo_ref[...], hhi_ref[...]], axis=1)
            a = jnp.concatenate([agg_ref[0], agg_ref[1]], axis=1)
            t = eps_ref[0, 0] * h + a
            return jnp.dot(t, w1_ref[...],
                           preferred_element_type=jnp.float32) + b1_ref[...]

        @pl.when(i < NBM)
        def _():
            u = compute_u()

            @pl.when(i == 0)
            def _():
                acc_u[...] = jnp.zeros_like(acc_u)
            acc_u[0:1, :] += jnp.sum(u, axis=0, keepdims=True)
            acc_u[1:2, :] += jnp.sum(u * u, axis=0, keepdims=True)

        @pl.when((i >= NBM) & (i < 2 * NBM))
        def _():
            j = i - NBM
            u = compute_u()
            mean = acc_u[0:1, :] * (1.0 / N)
            var = acc_u[1:2, :] * (1.0 / N) - mean * mean
            un = (u - mean) * (lax.rsqrt(var + 1e-5) * g1_ref[...]) + be1_ref[...]
            un = jnp.maximum(un, 0.0)
            v = jnp.dot(un, w2_ref[...],
                        preferred_element_type=jnp.float32) + b2_ref[...]
            v_s[pl.ds(j * RBM, RBM), :] = v

            @pl.when(j == 0)
            def _():
                acc_v[...] = jnp.zeros_like(acc_v)
            acc_v[0:1, :] += jnp.sum(v, axis=0, keepdims=True)
            acc_v[1:2, :] += jnp.sum(v * v, axis=0, keepdims=True)

        @pl.when(i >= 2 * NBM)
        def _():
            j = i - 2 * NBM
            v = v_s[pl.ds(j * RBM, RBM), :]
            mean = acc_v[0:1, :] * (1.0 / N)
            var = acc_v[1:2, :] * (1.0 / N) - mean * mean
            hn = (v - mean) * (lax.rsqrt(var + 1e-5) * g2_ref[...]) + be2_ref[...]
            hn = jnp.maximum(hn, 0.0)
            lo_ref[...] = hn[:, :HH]
            hi_ref[...] = hn[:, HH:]

    def clamp_ab(i):
        return jnp.where(i < 2 * NBM, lax.rem(i, NBM), 0)

    def clamp_c(i):
        return jnp.where(i >= 2 * NBM, i - 2 * NBM, 0)

    return pl.pallas_call(
        body,
        grid=(3 * NBM,),
        in_specs=[
            pl.BlockSpec((RBM, HH), lambda i: (clamp_ab(i), 0)),
            pl.BlockSpec((RBM, HH), lambda i: (clamp_ab(i), 0)),
            pl.BlockSpec((2, RBM, HH), lambda i: (0, clamp_ab(i), 0)),
            pl.BlockSpec((1, 1), lambda i: (0, 0)),
            pl.BlockSpec((H, H), lambda i: (0, 0)),
            pl.BlockSpec((1, H), lambda i: (0, 0)),
            pl.BlockSpec((1, H), lambda i: (0, 0)),
            pl.BlockSpec((1, H), lambda i: (0, 0)),
            pl.BlockSpec((H, H), lambda i: (0, 0)),
            pl.BlockSpec((1, H), lambda i: (0, 0)),
            pl.BlockSpec((1, H), lambda i: (0, 0)),
            pl.BlockSpec((1, H), lambda i: (0, 0)),
        ],
        out_specs=[pl.BlockSpec((RBM, HH), lambda i: (clamp_c(i), 0))] * 2,
        out_shape=[jax.ShapeDtypeStruct((N, HH), jnp.float32)] * 2,
        scratch_shapes=[
            pltpu.VMEM((N, H), jnp.float32),
            pltpu.VMEM((2, H), jnp.float32),
            pltpu.VMEM((2, H), jnp.float32),
        ],
    )(h_lo, h_hi, agg, (1.0 + eps_l).reshape(1, 1), W1_l, b1_l.reshape(1, H),
      g1_l.reshape(1, H), be1_l.reshape(1, H), W2_l, b2_l.reshape(1, H),
      g2_l.reshape(1, H), be2_l.reshape(1, H))


def _pool_cls(h_lo, h_hi, batch3, W_cls, b_cls):
    def body(lo_ref, hi_ref, b_ref, w_ref, bc_ref, o_ref, acc):
        i = pl.program_id(0)

        @pl.when(i == 0)
        def _():
            acc[...] = jnp.zeros_like(acc)

        h = jnp.concatenate(
            [lo_ref[...], hi_ref[...], jnp.ones((RB, 1), jnp.float32)], axis=1)
        bv = b_ref[0, 0, :]
        onehot = (bv[:, None] ==
                  lax.broadcasted_iota(jnp.int32, (RB, G), 1)).astype(jnp.float32)
        acc[...] += lax.dot_general(
            onehot, h, (((0,), (0,)), ((), ())),
            preferred_element_type=jnp.float32)

        @pl.when(i == NB - 1)
        def _():
            cnt = jnp.maximum(acc[:, H:H + 1], 1.0)
            pooled = acc[:, :H] / cnt
            o_ref[...] = jnp.dot(pooled, w_ref[...],
                                 preferred_element_type=jnp.float32) + bc_ref[...]

    return pl.pallas_call(
        body,
        grid=(NB,),
        in_specs=[
            pl.BlockSpec((RB, HH), lambda i: (i, 0)),
            pl.BlockSpec((RB, HH), lambda i: (i, 0)),
            pl.BlockSpec((1, 1, RB), lambda i: (i, 0, 0)),
            pl.BlockSpec((H, C), lambda i: (0, 0)),
            pl.BlockSpec((1, C), lambda i: (0, 0)),
        ],
        out_specs=pl.BlockSpec((G, C), lambda i: (0, 0)),
        out_shape=jax.ShapeDtypeStruct((G, C), jnp.float32),
        scratch_shapes=[pltpu.VMEM((G, H + 1), jnp.float32)],
    )(h_lo, h_hi, batch3, W_cls, b_cls.reshape(1, C))


def kernel(x, edge_index, batch, W_enc, b_enc, eps, W1, b1, g1, be1,
           W2, b2, g2, be2, W_cls, b_cls):
    src = edge_index[0].astype(jnp.int32)
    dst = edge_index[1].astype(jnp.int32)
    padn = EPAD - E
    src2 = jnp.concatenate([src, jnp.zeros((padn,), jnp.int32)]).reshape(ER, 128)
    dst2 = jnp.concatenate([dst, jnp.full((padn,), TRASH, jnp.int32)]).reshape(ER, 128)
    idxc = jnp.stack([src2, dst2], axis=1).reshape(2 * ER, 128)
    zblk = jnp.zeros((128, HH), jnp.float32)
    batch3 = batch.astype(jnp.int32).reshape(NB, 1, RB)

    h_lo, h_hi = _encoder(x, W_enc, b_enc)
    for l in range(L):
        agg = _sc_segment_sum(h_lo, h_hi, idxc, zblk)
        h_lo, h_hi = _gin_layer(h_lo, h_hi, agg, eps[l], W1[l], b1[l],
                                g1[l], be1[l], W2[l], b2[l], g2[l], be2[l])
    return _pool_cls(h_lo, h_hi, batch3, W_cls, b_cls)


# speedup vs baseline: 14.7927x; 1.0791x over previous
import functools

import jax
import jax.numpy as jnp
from jax import lax
from jax.experimental import pallas as pl
from jax.experimental.pallas import tpu as pltpu
from jax.experimental.pallas import tpu_sc as plsc

N = 50000
E = 1600000
H = 64
HH = 32
L = 3
G = 128
C = 2

RB = 2000
NB = N // RB
RBM = 5000
NBM = N // RBM

NPAD = 51200
TRASH = N
TILE_ROWS = NPAD // 16
PTR = 784
EPAD = PTR * 128 * 16
ER = EPAD // 128
CH = 2
NCH = PTR // CH


def _sc_segment_sum(h_lo, h_hi, idxc, zblk):
    mesh = plsc.VectorSubcoreMesh(core_axis_name="c", subcore_axis_name="s",
                                  num_cores=2, num_subcores=16)

    @functools.partial(
        pl.kernel,
        out_type=jax.ShapeDtypeStruct((2, NPAD, 128), jnp.float32),
        mesh=mesh,
        compiler_params=pltpu.CompilerParams(use_tc_tiling_on_sc=False),
        scratch_types=[
            pltpu.VMEM((6, 2 * CH, 128), jnp.int32),
            pltpu.VMEM((3, CH, 128, HH), jnp.float32),
            pltpu.VMEM_SHARED((NPAD, HH), jnp.float32),
            pltpu.SemaphoreType.DMA((6,)),
            pltpu.SemaphoreType.DMA((3,)),
            pltpu.SemaphoreType.DMA((3,)),
        ],
    )
    def k(hlo_hbm, hhi_hbm, idx_hbm, z_hbm, out_hbm,
          idx, rows, acc, isem, gsem, ssem):
        c = lax.axis_index("c")
        s = lax.axis_index("s")

        pltpu.sync_copy(z_hbm, rows.at[0, 0])

        def zloop(i, carry):
            pltpu.async_copy(rows.at[0, 0],
                             acc.at[pl.ds(s * TILE_ROWS + i * 128, 128)],
                             gsem.at[0])
            return carry
        lax.fori_loop(0, TILE_ROWS // 128, zloop, 0)

        def zdrain(i, carry):
            pltpu.make_async_copy(
                rows.at[0, 0], acc.at[pl.ds(s * TILE_ROWS + i * 128, 128)],
                gsem.at[0]).wait()
            return carry
        lax.fori_loop(0, TILE_ROWS // 128, zdrain, 0)
        plsc.subcore_barrier()

        def fire_idx(kc):
            q = lax.rem(kc, 6)
            r0 = 2 * (s * PTR + kc * CH)
            pltpu.async_copy(idx_hbm.at[pl.ds(r0, 2 * CH)], idx.at[q],
                             isem.at[q])

        def wait_idx(kc):
            q = lax.rem(kc, 6)
            r0 = 2 * (s * PTR + kc * CH)
            pltpu.make_async_copy(idx_hbm.at[pl.ds(r0, 2 * CH)], idx.at[q],
                                  isem.at[q]).wait()

        def fire_gathers(kc):
            p = lax.rem(kc, 3)
            q = lax.rem(kc, 6)

            def fj(j, carry):
                d_lo = pltpu.make_async_copy(
                    hlo_hbm.at[idx.at[q, 2 * j]], rows.at[p, j], gsem.at[p])
                d_hi = pltpu.make_async_copy(
                    hhi_hbm.at[idx.at[q, 2 * j]], rows.at[p, j], gsem.at[p])

                @pl.when(c == 0)
                def _():
                    d_lo.start()

                @pl.when(c == 1)
                def _():
                    d_hi.start()
                return carry
            lax.fori_loop(0, CH, fj, 0)

        def wait_gathers(kc):
            p = lax.rem(kc, 3)
            q = lax.rem(kc, 6)

            def wj(j, carry):
                pltpu.make_async_copy(
                    hlo_hbm.at[idx.at[q, 2 * j]], rows.at[p, j],
                    gsem.at[p]).wait()
                return carry
            lax.fori_loop(0, CH, wj, 0)

        def fire_scatters(kc):
            p = lax.rem(kc, 3)
            q = lax.rem(kc, 6)

            def sj(j, carry):
                pltpu.async_copy(rows.at[p, j], acc.at[idx.at[q, 2 * j + 1]],
                                 ssem.at[p], add=True)
                return carry
            lax.fori_loop(0, CH, sj, 0)

        def wait_scatters(kc):
            p = lax.rem(kc, 3)
            q = lax.rem(kc, 6)

            def wj(j, carry):
                pltpu.make_async_copy(rows.at[p, j], acc.at[idx.at[q, 2 * j + 1]],
                                      ssem.at[p]).wait()
                return carry
            lax.fori_loop(0, CH, wj, 0)

        lax.fori_loop(0, 4, lambda i, cr: (fire_idx(i), cr)[1], 0)

        def prol(i, cr):
            wait_idx(i)
            fire_gathers(i)
            return cr
        lax.fori_loop(0, 2, prol, 0)

        def body(kc, carry):
            @pl.when(kc >= 1)
            def _():
                wait_scatters(kc - 1)

            @pl.when(kc + 2 < NCH)
            def _():
                wait_idx(kc + 2)
                fire_gathers(kc + 2)

            wait_gathers(kc)
            fire_scatters(kc)

            @pl.when(kc + 4 < NCH)
            def _():
                fire_idx(kc + 4)
            return carry
        lax.fori_loop(0, NCH, body, 0)
        wait_scatters(NCH - 1)
        plsc.subcore_barrier()

        NWB = TILE_ROWS // 128

        def rd(i):
            p = lax.rem(i, 3)
            pltpu.async_copy(acc.at[pl.ds(s * TILE_ROWS + i * 128, 128)],
                             rows.at[p, 0], gsem.at[p])

        def rd_wait(i):
            p = lax.rem(i, 3)
            pltpu.make_async_copy(
                acc.at[pl.ds(s * TILE_ROWS + i * 128, 128)], rows.at[p, 0],
                gsem.at[p]).wait()

        def wr(i):
            p = lax.rem(i, 3)
            pltpu.async_copy(
                rows.at[p, 0],
                out_hbm.at[c, pl.ds(s * TILE_ROWS + i * 128, 128),
                           pl.ds(0, HH)],
                ssem.at[p])

        def wr_wait(i):
            p = lax.rem(i, 3)
            pltpu.make_async_copy(
                rows.at[p, 0],
                out_hbm.at[c, pl.ds(s * TILE_ROWS + i * 128, 128),
                           pl.ds(0, HH)],
                ssem.at[p]).wait()

        rd(0)
        rd(1)

        def wloop(i, carry):
            @pl.when(i >= 1)
            def _():
                wr_wait(i - 1)

            rd_wait(i)
            wr(i)

            @pl.when(i + 2 < NWB)
            def _():
                rd(i + 2)
            return carry
        lax.fori_loop(0, NWB, wloop, 0)
        wr_wait(NWB - 1)

    return k(h_lo, h_hi, idxc, zblk)


def _encoder(x, W_enc, b_enc):
    def body(x_ref, w_ref, b_ref, lo_ref, hi_ref):
        hblk = x_ref[...] * w_ref[...] + b_ref[...]
        lo_ref[...] = hblk[:, :HH]
        hi_ref[...] = hblk[:, HH:]

    return pl.pallas_call(
        body,
        grid=(NB,),
        in_specs=[
            pl.BlockSpec((RB, 1), lambda i: (i, 0)),
            pl.BlockSpec((1, H), lambda i: (0, 0)),
            pl.BlockSpec((1, H), lambda i: (0, 0)),
        ],
        out_specs=[pl.BlockSpec((RB, HH), lambda i: (i, 0))] * 2,
        out_shape=[jax.ShapeDtypeStruct((N, HH), jnp.float32)] * 2,
    )(x, W_enc, b_enc.reshape(1, H))


def _gin_layer(h_lo, h_hi, agg, eps_l, W1_l, b1_l, g1_l, be1_l,
               W2_l, b2_l, g2_l, be2_l):
    def body(hlo_ref, hhi_ref, agg_ref, eps_ref, w1_ref, b1_ref, g1_ref,
             be1_ref, w2_ref, b2_ref, g2_ref, be2_ref, lo_ref, hi_ref,
             v_s, acc_u, acc_v):
        i = pl.program_id(0)

        def compute_u():
            h = jnp.concatenate([hlo_ref[...], hhi_ref[...]], axis=1)
            a = jnp.concatenate(
                [agg_ref[0][:, :HH], agg_ref[1][:, :HH]], axis=1)
            t = eps_ref[0, 0] * h + a
            return jnp.dot(t, w1_ref[...],
                           preferred_element_type=jnp.float32) + b1_ref[...]

        @pl.when(i < NBM)
        def _():
            u = compute_u()

            @pl.when(i == 0)
            def _():
                acc_u[...] = jnp.zeros_like(acc_u)
            acc_u[0:1, :] += jnp.sum(u, axis=0, keepdims=True)
            acc_u[1:2, :] += jnp.sum(u * u, axis=0, keepdims=True)

        @pl.when((i >= NBM) & (i < 2 * NBM))
        def _():
            j = i - NBM
            u = compute_u()
            mean = acc_u[0:1, :] * (1.0 / N)
            var = acc_u[1:2, :] * (1.0 / N) - mean * mean
            un = (u - mean) * (lax.rsqrt(var + 1e-5) * g1_ref[...]) + be1_ref[...]
            un = jnp.maximum(un, 0.0)
            v = jnp.dot(un, w2_ref[...],
                        preferred_element_type=jnp.float32) + b2_ref[...]
            v_s[pl.ds(j * RBM, RBM), :] = v

            @pl.when(j == 0)
            def _():
                acc_v[...] = jnp.zeros_like(acc_v)
            acc_v[0:1, :] += jnp.sum(v, axis=0, keepdims=True)
            acc_v[1:2, :] += jnp.sum(v * v, axis=0, keepdims=True)

        @pl.when(i >= 2 * NBM)
        def _():
            j = i - 2 * NBM
            v = v_s[pl.ds(j * RBM, RBM), :]
            mean = acc_v[0:1, :] * (1.0 / N)
            var = acc_v[1:2, :] * (1.0 / N) - mean * mean
            hn = (v - mean) * (lax.rsqrt(var + 1e-5) * g2_ref[...]) + be2_ref[...]
            hn = jnp.maximum(hn, 0.0)
            lo_ref[...] = hn[:, :HH]
            hi_ref[...] = hn[:, HH:]

    def clamp_ab(i):
        return jnp.where(i < 2 * NBM, lax.rem(i, NBM), 0)

    def clamp_c(i):
        return jnp.where(i >= 2 * NBM, i - 2 * NBM, 0)

    return pl.pallas_call(
        body,
        grid=(3 * NBM,),
        in_specs=[
            pl.BlockSpec((RBM, HH), lambda i: (clamp_ab(i), 0)),
            pl.BlockSpec((RBM, HH), lambda i: (clamp_ab(i), 0)),
            pl.BlockSpec((2, RBM, 128), lambda i: (0, clamp_ab(i), 0)),
            pl.BlockSpec((1, 1), lambda i: (0, 0)),
            pl.BlockSpec((H, H), lambda i: (0, 0)),
            pl.BlockSpec((1, H), lambda i: (0, 0)),
            pl.BlockSpec((1, H), lambda i: (0, 0)),
            pl.BlockSpec((1, H), lambda i: (0, 0)),
            pl.BlockSpec((H, H), lambda i: (0, 0)),
            pl.BlockSpec((1, H), lambda i: (0, 0)),
            pl.BlockSpec((1, H), lambda i: (0, 0)),
            pl.BlockSpec((1, H), lambda i: (0, 0)),
        ],
        out_specs=[pl.BlockSpec((RBM, HH), lambda i: (clamp_c(i), 0))] * 2,
        out_shape=[jax.ShapeDtypeStruct((N, HH), jnp.float32)] * 2,
        scratch_shapes=[
            pltpu.VMEM((N, H), jnp.float32),
            pltpu.VMEM((2, H), jnp.float32),
            pltpu.VMEM((2, H), jnp.float32),
        ],
    )(h_lo, h_hi, agg, (1.0 + eps_l).reshape(1, 1), W1_l, b1_l.reshape(1, H),
      g1_l.reshape(1, H), be1_l.reshape(1, H), W2_l, b2_l.reshape(1, H),
      g2_l.reshape(1, H), be2_l.reshape(1, H))


def _pool_cls(h_lo, h_hi, batch3, W_cls, b_cls):
    def body(lo_ref, hi_ref, b_ref, w_ref, bc_ref, o_ref, acc):
        i = pl.program_id(0)

        @pl.when(i == 0)
        def _():
            acc[...] = jnp.zeros_like(acc)

        h = jnp.concatenate(
            [lo_ref[...], hi_ref[...], jnp.ones((RB, 1), jnp.float32)], axis=1)
        bv = b_ref[0, 0, :]
        onehot = (bv[:, None] ==
                  lax.broadcasted_iota(jnp.int32, (RB, G), 1)).astype(jnp.float32)
        acc[...] += lax.dot_general(
            onehot, h, (((0,), (0,)), ((), ())),
            preferred_element_type=jnp.float32)

        @pl.when(i == NB - 1)
        def _():
            cnt = jnp.maximum(acc[:, H:H + 1], 1.0)
            pooled = acc[:, :H] / cnt
            o_ref[...] = jnp.dot(pooled, w_ref[...],
                                 preferred_element_type=jnp.float32) + bc_ref[...]

    return pl.pallas_call(
        body,
        grid=(NB,),
        in_specs=[
            pl.BlockSpec((RB, HH), lambda i: (i, 0)),
            pl.BlockSpec((RB, HH), lambda i: (i, 0)),
            pl.BlockSpec((1, 1, RB), lambda i: (i, 0, 0)),
            pl.BlockSpec((H, C), lambda i: (0, 0)),
            pl.BlockSpec((1, C), lambda i: (0, 0)),
        ],
        out_specs=pl.BlockSpec((G, C), lambda i: (0, 0)),
        out_shape=jax.ShapeDtypeStruct((G, C), jnp.float32),
        scratch_shapes=[pltpu.VMEM((G, H + 1), jnp.float32)],
    )(h_lo, h_hi, batch3, W_cls, b_cls.reshape(1, C))


def kernel(x, edge_index, batch, W_enc, b_enc, eps, W1, b1, g1, be1,
           W2, b2, g2, be2, W_cls, b_cls):
    src = edge_index[0].astype(jnp.int32)
    dst = edge_index[1].astype(jnp.int32)
    padn = EPAD - E
    src2 = jnp.concatenate([src, jnp.zeros((padn,), jnp.int32)]).reshape(ER, 128)
    dst2 = jnp.concatenate([dst, jnp.full((padn,), TRASH, jnp.int32)]).reshape(ER, 128)
    idxc = jnp.stack([src2, dst2], axis=1).reshape(2 * ER, 128)
    zblk = jnp.zeros((128, HH), jnp.float32)
    batch3 = batch.astype(jnp.int32).reshape(NB, 1, RB)

    h_lo, h_hi = _encoder(x, W_enc, b_enc)
    for l in range(L):
        agg = _sc_segment_sum(h_lo, h_hi, idxc, zblk)
        h_lo, h_hi = _gin_layer(h_lo, h_hi, agg, eps[l], W1[l], b1[l],
                                g1[l], be1[l], W2[l], b2[l], g2[l], be2[l])
    return _pool_cls(h_lo, h_hi, batch3, W_cls, b_cls)
